# baseline (speedup 1.0000x reference)
import functools

import jax
import jax.numpy as jnp
from jax import lax
from jax.experimental import pallas as pl
from jax.experimental.pallas import tpu as pltpu
from jax.experimental.pallas import tpu_sc as plsc

Z_DIM = 1_000_000
FEAT = 16
NC = 2
NS = 16
NW = NC * NS

BLK = 1024
NFULL = Z_DIM // BLK
TAIL = Z_DIM - NFULL * BLK
TAIL_A = 512
TAIL_B = 64
NB_LO = NFULL // NW
NB_REM = NFULL - NB_LO * NW

CHUNK = 2000
NCHUNK = Z_DIM // CHUNK
BASE_PER_W = NCHUNK // NW
REM = NCHUNK - BASE_PER_W * NW


def _worker_id():
    return lax.axis_index("s") * NC + lax.axis_index("c")


def _shuffle_detile(t0, t1, rv, ncols):
    iota = lax.broadcasted_iota(jnp.int32, (16,), 0)

    def body(g, carry):
        c0 = g * 16
        cvec = c0 + iota
        d0 = lax.shift_right_logical(cvec, 3)
        tbase = lax.shift_left(lax.bitwise_and(cvec, 7), 4)
        vals0 = [t0[j, pl.ds(c0, 16)] for j in range(8)]
        vals1 = [t1[j, pl.ds(c0, 16)] for j in range(8)]
        for j in range(8):
            plsc.store_scatter(rv, [d0, tbase + j], vals0[j])
        for j in range(8):
            plsc.store_scatter(rv, [d0, tbase + (j + 8)], vals1[j])
        return carry

    lax.fori_loop(0, ncols // 16, body, 0, unroll=2)


def _shuffle_retile(u, rv, ncols):
    iota = lax.broadcasted_iota(jnp.int32, (16,), 0)

    def body(g, carry):
        c0 = g * 16
        cvec = c0 + iota
        d0 = lax.shift_right_logical(cvec, 3)
        tbase = lax.shift_left(lax.bitwise_and(cvec, 7), 4)
        vals = [plsc.load_gather(rv, [d0, tbase + j]) for j in range(16)]
        for j in range(16):
            u[j, pl.ds(c0, 16)] = vals[j]
        return carry

    lax.fori_loop(0, ncols // 16, body, 0, unroll=2)


def _detile_body(zt_hbm, out_hbm, t0a, t1a, rva, t0b, t1b, rvb,
                 sia, sib, soa, sob):
    w = _worker_id()
    nb = jnp.where(w < NB_REM, NB_LO + 1, NB_LO)

    def issue_in(cb, t0, t1, sem):
        col = cb * BLK
        pltpu.async_copy(zt_hbm.at[pl.ds(0, 8), pl.ds(col, BLK)], t0, sem)
        pltpu.async_copy(zt_hbm.at[pl.ds(8, 8), pl.ds(col, BLK)], t1, sem)

    def wait_in(t0, t1, sem):
        pltpu.make_async_copy(zt_hbm.at[pl.ds(0, 8), pl.ds(0, BLK)], t0, sem).wait()
        pltpu.make_async_copy(zt_hbm.at[pl.ds(8, 8), pl.ds(0, BLK)], t1, sem).wait()

    def issue_out(cb, rv, sem):
        pltpu.async_copy(rv, out_hbm.at[pl.ds(cb * (BLK // 8), BLK // 8)], sem)

    def wait_out(rv, sem):
        pltpu.make_async_copy(rv, out_hbm.at[pl.ds(0, BLK // 8)], sem).wait()

    @pl.when(nb > 0)
    def _():
        issue_in(w, t0a, t1a, sia)

        def pair(k, carry):
            i0 = 2 * k
            cb0 = w + 32 * i0

            @pl.when(i0 + 1 < nb)
            def _():
                issue_in(w + 32 * (i0 + 1), t0b, t1b, sib)

            wait_in(t0a, t1a, sia)
            _shuffle_detile(t0a, t1a, rva, BLK)

            @pl.when(k > 0)
            def _():
                wait_out(rva, soa)

            issue_out(cb0, rva, soa)

            @pl.when(i0 + 1 < nb)
            def _():
                @pl.when(i0 + 2 < nb)
                def _():
                    issue_in(w + 32 * (i0 + 2), t0a, t1a, sia)

                wait_in(t0b, t1b, sib)
                _shuffle_detile(t0b, t1b, rvb, BLK)

                @pl.when(k > 0)
                def _():
                    wait_out(rvb, sob)

                issue_out(w + 32 * (i0 + 1), rvb, sob)

            return carry

        npairs = (nb + 1) // 2
        lax.fori_loop(0, npairs, pair, 0)
        wait_out(rva, soa)

        @pl.when(nb > 1)
        def _():
            wait_out(rvb, sob)

    @pl.when(w == 30)
    def _():
        col = NFULL * BLK
        pltpu.async_copy(zt_hbm.at[pl.ds(0, 8), pl.ds(col, TAIL_A)],
                         t0a.at[:, pl.ds(0, TAIL_A)], sia)
        pltpu.async_copy(zt_hbm.at[pl.ds(8, 8), pl.ds(col, TAIL_A)],
                         t1a.at[:, pl.ds(0, TAIL_A)], sia)
        pltpu.make_async_copy(zt_hbm.at[pl.ds(0, 8), pl.ds(0, TAIL_A)],
                              t0a.at[:, pl.ds(0, TAIL_A)], sia).wait()
        pltpu.make_async_copy(zt_hbm.at[pl.ds(8, 8), pl.ds(0, TAIL_A)],
                              t1a.at[:, pl.ds(0, TAIL_A)], sia).wait()
        _shuffle_detile(t0a, t1a, rva, TAIL_A)
        pltpu.async_copy(rva.at[pl.ds(0, TAIL_A // 8)],
                         out_hbm.at[pl.ds(col // 8, TAIL_A // 8)], soa)
        pltpu.make_async_copy(rva.at[pl.ds(0, TAIL_A // 8)],
                              out_hbm.at[pl.ds(0, TAIL_A // 8)], soa).wait()

    @pl.when(w == 31)
    def _():
        col = pl.multiple_of(jnp.int32(Z_DIM - 64), 128)
        colw = Z_DIM - 64
        pltpu.async_copy(zt_hbm.at[pl.ds(0, 8), pl.ds(col, 128)],
                         t0a.at[:, pl.ds(0, 128)], sia)
        pltpu.async_copy(zt_hbm.at[pl.ds(8, 8), pl.ds(col, 128)],
                         t1a.at[:, pl.ds(0, 128)], sia)
        pltpu.make_async_copy(zt_hbm.at[pl.ds(0, 8), pl.ds(0, 128)],
                              t0a.at[:, pl.ds(0, 128)], sia).wait()
        pltpu.make_async_copy(zt_hbm.at[pl.ds(8, 8), pl.ds(0, 128)],
                              t1a.at[:, pl.ds(0, 128)], sia).wait()
        _shuffle_detile(t0a, t1a, rva, 128)
        pltpu.async_copy(rva.at[pl.ds(0, 8)],
                         out_hbm.at[pl.ds(colw // 8, 8)], soa)
        pltpu.make_async_copy(rva.at[pl.ds(0, 8)],
                              out_hbm.at[pl.ds(0, 8)], soa).wait()


def _retile_body(rows_hbm, out_hbm, ua, rva, ub, rvb,
                 sia, sib, soa, sob):
    w = _worker_id()
    nb = jnp.where(w < NB_REM, NB_LO + 1, NB_LO)

    def issue_in(cb, rv, sem):
        pltpu.async_copy(rows_hbm.at[pl.ds(cb * (BLK // 8), BLK // 8)], rv, sem)

    def wait_in(rv, sem):
        pltpu.make_async_copy(rows_hbm.at[pl.ds(0, BLK // 8)], rv, sem).wait()

    def issue_out(cb, u, sem):
        col = cb * BLK
        pltpu.async_copy(u.at[pl.ds(0, 8)], out_hbm.at[pl.ds(0, 8), pl.ds(col, BLK)], sem)
        pltpu.async_copy(u.at[pl.ds(8, 8)], out_hbm.at[pl.ds(8, 8), pl.ds(col, BLK)], sem)

    def wait_out(u, sem):
        pltpu.make_async_copy(u.at[pl.ds(0, 8)], out_hbm.at[pl.ds(0, 8), pl.ds(0, BLK)], sem).wait()
        pltpu.make_async_copy(u.at[pl.ds(8, 8)], out_hbm.at[pl.ds(8, 8), pl.ds(0, BLK)], sem).wait()

    @pl.when(nb > 0)
    def _():
        issue_in(w, rva, sia)

        def pair(k, carry):
            i0 = 2 * k
            cb0 = w + 32 * i0

            @pl.when(i0 + 1 < nb)
            def _():
                issue_in(w + 32 * (i0 + 1), rvb, sib)

            wait_in(rva, sia)

            @pl.when(k > 0)
            def _():
                wait_out(ua, soa)

            _shuffle_retile(ua, rva, BLK)
            issue_out(cb0, ua, soa)

            @pl.when(i0 + 1 < nb)
            def _():
                @pl.when(i0 + 2 < nb)
                def _():
                    issue_in(w + 32 * (i0 + 2), rva, sia)

                wait_in(rvb, sib)

                @pl.when(k > 0)
                def _():
                    wait_out(ub, sob)

                _shuffle_retile(ub, rvb, BLK)
                issue_out(w + 32 * (i0 + 1), ub, sob)

            return carry

        npairs = (nb + 1) // 2
        lax.fori_loop(0, npairs, pair, 0)
        wait_out(ua, soa)

        @pl.when(nb > 1)
        def _():
            wait_out(ub, sob)

    @pl.when(w == 30)
    def _():
        col = NFULL * BLK
        pltpu.async_copy(rows_hbm.at[pl.ds(col // 8, TAIL_A // 8)],
                         rva.at[pl.ds(0, TAIL_A // 8)], sia)
        pltpu.make_async_copy(rows_hbm.at[pl.ds(0, TAIL_A // 8)],
                              rva.at[pl.ds(0, TAIL_A // 8)], sia).wait()
        _shuffle_retile(ua, rva, TAIL_A)
        pltpu.async_copy(ua.at[pl.ds(0, 8), pl.ds(0, TAIL_A)],
                         out_hbm.at[pl.ds(0, 8), pl.ds(col, TAIL_A)], soa)
        pltpu.async_copy(ua.at[pl.ds(8, 8), pl.ds(0, TAIL_A)],
                         out_hbm.at[pl.ds(8, 8), pl.ds(col, TAIL_A)], soa)
        pltpu.make_async_copy(ua.at[pl.ds(0, 8), pl.ds(0, TAIL_A)],
                              out_hbm.at[pl.ds(0, 8), pl.ds(0, TAIL_A)], soa).wait()
        pltpu.make_async_copy(ua.at[pl.ds(8, 8), pl.ds(0, TAIL_A)],
                              out_hbm.at[pl.ds(8, 8), pl.ds(0, TAIL_A)], soa).wait()

    @pl.when(w == 31)
    def _():
        colw = Z_DIM - 64
        col = pl.multiple_of(jnp.int32(Z_DIM - 64), 128)
        pltpu.async_copy(rows_hbm.at[pl.ds(colw // 8, 8)],
                         rva.at[pl.ds(0, 8)], sia)
        pltpu.make_async_copy(rows_hbm.at[pl.ds(0, 8)],
                              rva.at[pl.ds(0, 8)], sia).wait()
        _shuffle_retile(ua, rva, 128)
        pltpu.async_copy(ua.at[pl.ds(0, 8), pl.ds(0, 128)],
                         out_hbm.at[pl.ds(0, 8), pl.ds(col, 128)], soa)
        pltpu.async_copy(ua.at[pl.ds(8, 8), pl.ds(0, 128)],
                         out_hbm.at[pl.ds(8, 8), pl.ds(col, 128)], soa)
        pltpu.make_async_copy(ua.at[pl.ds(0, 8), pl.ds(0, 128)],
                              out_hbm.at[pl.ds(0, 8), pl.ds(0, 128)], soa).wait()
        pltpu.make_async_copy(ua.at[pl.ds(8, 8), pl.ds(0, 128)],
                              out_hbm.at[pl.ds(8, 8), pl.ds(0, 128)], soa).wait()


def _gather_body(z_hbm, p_hbm, out_hbm, ia, ra, ib, rb,
                 siia, sga, sooa, siib, sgb, soob):
    w = _worker_id()
    start = BASE_PER_W * w + jnp.minimum(w, REM)
    n = jnp.where(w < REM, BASE_PER_W + 1, BASE_PER_W)

    def issue_idx(c, iv, sem):
        pltpu.async_copy(p_hbm.at[pl.ds(c * CHUNK, CHUNK)], iv, sem)

    def wait_idx(iv, sem):
        pltpu.make_async_copy(p_hbm.at[pl.ds(0, CHUNK)], iv, sem).wait()

    def issue_g(iv, rv, sem):
        pltpu.async_copy(z_hbm.at[iv], rv, sem)

    def wait_g(iv, rv, sem):
        pltpu.make_async_copy(z_hbm.at[iv], rv, sem).wait()

    def issue_out(c, rv, sem):
        pltpu.async_copy(rv, out_hbm.at[pl.ds(c * CHUNK, CHUNK)], sem)

    def wait_out(rv, sem):
        pltpu.make_async_copy(rv, out_hbm.at[pl.ds(0, CHUNK)], sem).wait()

    issue_idx(start, ia, siia)

    def pair(k, carry):
        i0 = 2 * k
        c0 = start + i0

        @pl.when(i0 + 1 < n)
        def _():
            issue_idx(c0 + 1, ib, siib)

        wait_idx(ia, siia)

        @pl.when(k > 0)
        def _():
            wait_out(ra, sooa)

        issue_g(ia, ra, sga)

        @pl.when(i0 + 1 < n)
        def _():
            wait_idx(ib, siib)

            @pl.when(k > 0)
            def _():
                wait_out(rb, soob)

            issue_g(ib, rb, sgb)

        wait_g(ia, ra, sga)
        issue_out(c0, ra, sooa)

        @pl.when(i0 + 2 < n)
        def _():
            issue_idx(c0 + 2, ia, siia)

        @pl.when(i0 + 1 < n)
        def _():
            wait_g(ib, rb, sgb)
            issue_out(c0 + 1, rb, soob)

        return carry

    lax.fori_loop(0, (n + 1) // 2, pair, 0)
    wait_out(ra, sooa)

    @pl.when(n > 1)
    def _():
        wait_out(rb, soob)


def _mesh():
    return plsc.VectorSubcoreMesh(
        core_axis_name="c", subcore_axis_name="s", num_cores=NC, num_subcores=NS
    )


def _tile_scratch():
    return [
        pltpu.VMEM((8, BLK), jnp.float32),
        pltpu.VMEM((8, BLK), jnp.float32),
        pltpu.VMEM((BLK // 8, 128), jnp.float32),
        pltpu.VMEM((8, BLK), jnp.float32),
        pltpu.VMEM((8, BLK), jnp.float32),
        pltpu.VMEM((BLK // 8, 128), jnp.float32),
        pltpu.SemaphoreType.DMA,
        pltpu.SemaphoreType.DMA,
        pltpu.SemaphoreType.DMA,
        pltpu.SemaphoreType.DMA,
    ]


@jax.jit
def _permute_gather(z, permute):
    zt = z.T

    z_rows128 = pl.kernel(
        _detile_body,
        out_type=jax.ShapeDtypeStruct((Z_DIM // 8, 128), jnp.float32),
        mesh=_mesh(),
        scratch_types=_tile_scratch(),
        compiler_params=pltpu.CompilerParams(use_tc_tiling_on_sc=True, needs_layout_passes=False, disable_bounds_checks=True),
    )(zt)

    out_lin = pl.kernel(
        _gather_body,
        out_type=jax.ShapeDtypeStruct((Z_DIM, FEAT), jnp.float32),
        mesh=_mesh(),
        scratch_types=[
            pltpu.VMEM((CHUNK,), jnp.int32),
            pltpu.VMEM((CHUNK, FEAT), jnp.float32),
            pltpu.VMEM((CHUNK,), jnp.int32),
            pltpu.VMEM((CHUNK, FEAT), jnp.float32),
            pltpu.SemaphoreType.DMA,
            pltpu.SemaphoreType.DMA,
            pltpu.SemaphoreType.DMA,
            pltpu.SemaphoreType.DMA,
            pltpu.SemaphoreType.DMA,
            pltpu.SemaphoreType.DMA,
        ],
        compiler_params=pltpu.CompilerParams(use_tc_tiling_on_sc=False),
    )(z_rows128.reshape(Z_DIM, FEAT), permute)

    out_t = pl.kernel(
        _retile_body,
        out_type=jax.ShapeDtypeStruct((FEAT, Z_DIM), jnp.float32),
        mesh=_mesh(),
        scratch_types=[
            pltpu.VMEM((FEAT, BLK), jnp.float32),
            pltpu.VMEM((BLK // 8, 128), jnp.float32),
            pltpu.VMEM((FEAT, BLK), jnp.float32),
            pltpu.VMEM((BLK // 8, 128), jnp.float32),
            pltpu.SemaphoreType.DMA,
            pltpu.SemaphoreType.DMA,
            pltpu.SemaphoreType.DMA,
            pltpu.SemaphoreType.DMA,
        ],
        compiler_params=pltpu.CompilerParams(use_tc_tiling_on_sc=True, needs_layout_passes=False, disable_bounds_checks=True),
    )(out_lin.reshape(Z_DIM // 8, 128))

    return out_t.T


def kernel(z, permute):
    return _permute_gather(z, permute.astype(jnp.int32))


# speedup vs baseline: 1.0167x; 1.0167x over previous
import functools

import jax
import jax.numpy as jnp
from jax import lax
from jax.experimental import pallas as pl
from jax.experimental.pallas import tpu as pltpu
from jax.experimental.pallas import tpu_sc as plsc

Z_DIM = 1_000_000
FEAT = 16
NC = 2
NS = 16
NW = NC * NS

BLK = 1024
NFULL = Z_DIM // BLK
TAIL = Z_DIM - NFULL * BLK
TAIL_A = 512
TAIL_B = 64
NB_LO = NFULL // NW
NB_REM = NFULL - NB_LO * NW

CHUNK = 2000
NCHUNK = Z_DIM // CHUNK
BASE_PER_W = NCHUNK // NW
REM = NCHUNK - BASE_PER_W * NW


def _worker_id():
    return lax.axis_index("s") * NC + lax.axis_index("c")


def _shuffle_detile(t0, t1, rv, ncols):
    iota = lax.broadcasted_iota(jnp.int32, (16,), 0)

    def body(g, carry):
        c0 = g * 16
        cvec = c0 + iota
        d0 = lax.shift_right_logical(cvec, 3)
        tbase = lax.shift_left(lax.bitwise_and(cvec, 7), 4)
        vals0 = [t0[j, pl.ds(c0, 16)] for j in range(8)]
        vals1 = [t1[j, pl.ds(c0, 16)] for j in range(8)]
        for j in range(8):
            plsc.store_scatter(rv, [d0, tbase + j], vals0[j])
        for j in range(8):
            plsc.store_scatter(rv, [d0, tbase + (j + 8)], vals1[j])
        return carry

    lax.fori_loop(0, ncols // 16, body, 0, unroll=2)


def _shuffle_retile(u, rv, ncols):
    iota = lax.broadcasted_iota(jnp.int32, (16,), 0)

    def body(g, carry):
        c0 = g * 16
        cvec = c0 + iota
        d0 = lax.shift_right_logical(cvec, 3)
        tbase = lax.shift_left(lax.bitwise_and(cvec, 7), 4)
        vals = [plsc.load_gather(rv, [d0, tbase + j]) for j in range(16)]
        for j in range(16):
            u[j, pl.ds(c0, 16)] = vals[j]
        return carry

    lax.fori_loop(0, ncols // 16, body, 0, unroll=4)


def _detile_body(zt_hbm, out_hbm, t0a, t1a, rva, t0b, t1b, rvb,
                 sia, sib, soa, sob):
    w = _worker_id()
    nb = jnp.where(w < NB_REM, NB_LO + 1, NB_LO)

    def issue_in(cb, t0, t1, sem):
        col = cb * BLK
        pltpu.async_copy(zt_hbm.at[pl.ds(0, 8), pl.ds(col, BLK)], t0, sem)
        pltpu.async_copy(zt_hbm.at[pl.ds(8, 8), pl.ds(col, BLK)], t1, sem)

    def wait_in(t0, t1, sem):
        pltpu.make_async_copy(zt_hbm.at[pl.ds(0, 8), pl.ds(0, BLK)], t0, sem).wait()
        pltpu.make_async_copy(zt_hbm.at[pl.ds(8, 8), pl.ds(0, BLK)], t1, sem).wait()

    def issue_out(cb, rv, sem):
        pltpu.async_copy(rv, out_hbm.at[pl.ds(cb * (BLK // 8), BLK // 8)], sem)

    def wait_out(rv, sem):
        pltpu.make_async_copy(rv, out_hbm.at[pl.ds(0, BLK // 8)], sem).wait()

    @pl.when(nb > 0)
    def _():
        issue_in(w, t0a, t1a, sia)

        def pair(k, carry):
            i0 = 2 * k
            cb0 = w + 32 * i0

            @pl.when(i0 + 1 < nb)
            def _():
                issue_in(w + 32 * (i0 + 1), t0b, t1b, sib)

            wait_in(t0a, t1a, sia)
            _shuffle_detile(t0a, t1a, rva, BLK)

            @pl.when(k > 0)
            def _():
                wait_out(rva, soa)

            issue_out(cb0, rva, soa)

            @pl.when(i0 + 1 < nb)
            def _():
                @pl.when(i0 + 2 < nb)
                def _():
                    issue_in(w + 32 * (i0 + 2), t0a, t1a, sia)

                wait_in(t0b, t1b, sib)
                _shuffle_detile(t0b, t1b, rvb, BLK)

                @pl.when(k > 0)
                def _():
                    wait_out(rvb, sob)

                issue_out(w + 32 * (i0 + 1), rvb, sob)

            return carry

        npairs = (nb + 1) // 2
        lax.fori_loop(0, npairs, pair, 0)
        wait_out(rva, soa)

        @pl.when(nb > 1)
        def _():
            wait_out(rvb, sob)

    @pl.when(w == 30)
    def _():
        col = NFULL * BLK
        pltpu.async_copy(zt_hbm.at[pl.ds(0, 8), pl.ds(col, TAIL_A)],
                         t0a.at[:, pl.ds(0, TAIL_A)], sia)
        pltpu.async_copy(zt_hbm.at[pl.ds(8, 8), pl.ds(col, TAIL_A)],
                         t1a.at[:, pl.ds(0, TAIL_A)], sia)
        pltpu.make_async_copy(zt_hbm.at[pl.ds(0, 8), pl.ds(0, TAIL_A)],
                              t0a.at[:, pl.ds(0, TAIL_A)], sia).wait()
        pltpu.make_async_copy(zt_hbm.at[pl.ds(8, 8), pl.ds(0, TAIL_A)],
                              t1a.at[:, pl.ds(0, TAIL_A)], sia).wait()
        _shuffle_detile(t0a, t1a, rva, TAIL_A)
        pltpu.async_copy(rva.at[pl.ds(0, TAIL_A // 8)],
                         out_hbm.at[pl.ds(col // 8, TAIL_A // 8)], soa)
        pltpu.make_async_copy(rva.at[pl.ds(0, TAIL_A // 8)],
                              out_hbm.at[pl.ds(0, TAIL_A // 8)], soa).wait()

    @pl.when(w == 31)
    def _():
        col = pl.multiple_of(jnp.int32(Z_DIM - 64), 128)
        colw = Z_DIM - 64
        pltpu.async_copy(zt_hbm.at[pl.ds(0, 8), pl.ds(col, 128)],
                         t0a.at[:, pl.ds(0, 128)], sia)
        pltpu.async_copy(zt_hbm.at[pl.ds(8, 8), pl.ds(col, 128)],
                         t1a.at[:, pl.ds(0, 128)], sia)
        pltpu.make_async_copy(zt_hbm.at[pl.ds(0, 8), pl.ds(0, 128)],
                              t0a.at[:, pl.ds(0, 128)], sia).wait()
        pltpu.make_async_copy(zt_hbm.at[pl.ds(8, 8), pl.ds(0, 128)],
                              t1a.at[:, pl.ds(0, 128)], sia).wait()
        _shuffle_detile(t0a, t1a, rva, 128)
        pltpu.async_copy(rva.at[pl.ds(0, 8)],
                         out_hbm.at[pl.ds(colw // 8, 8)], soa)
        pltpu.make_async_copy(rva.at[pl.ds(0, 8)],
                              out_hbm.at[pl.ds(0, 8)], soa).wait()


def _retile_body(rows_hbm, out_hbm, ua, rva, ub, rvb,
                 sia, sib, soa, sob):
    w = _worker_id()
    nb = jnp.where(w < NB_REM, NB_LO + 1, NB_LO)

    def issue_in(cb, rv, sem):
        pltpu.async_copy(rows_hbm.at[pl.ds(cb * (BLK // 8), BLK // 8)], rv, sem)

    def wait_in(rv, sem):
        pltpu.make_async_copy(rows_hbm.at[pl.ds(0, BLK // 8)], rv, sem).wait()

    def issue_out(cb, u, sem):
        col = cb * BLK
        pltpu.async_copy(u.at[pl.ds(0, 8)], out_hbm.at[pl.ds(0, 8), pl.ds(col, BLK)], sem)
        pltpu.async_copy(u.at[pl.ds(8, 8)], out_hbm.at[pl.ds(8, 8), pl.ds(col, BLK)], sem)

    def wait_out(u, sem):
        pltpu.make_async_copy(u.at[pl.ds(0, 8)], out_hbm.at[pl.ds(0, 8), pl.ds(0, BLK)], sem).wait()
        pltpu.make_async_copy(u.at[pl.ds(8, 8)], out_hbm.at[pl.ds(8, 8), pl.ds(0, BLK)], sem).wait()

    @pl.when(nb > 0)
    def _():
        issue_in(w, rva, sia)

        def pair(k, carry):
            i0 = 2 * k
            cb0 = w + 32 * i0

            @pl.when(i0 + 1 < nb)
            def _():
                issue_in(w + 32 * (i0 + 1), rvb, sib)

            wait_in(rva, sia)

            @pl.when(k > 0)
            def _():
                wait_out(ua, soa)

            _shuffle_retile(ua, rva, BLK)
            issue_out(cb0, ua, soa)

            @pl.when(i0 + 1 < nb)
            def _():
                @pl.when(i0 + 2 < nb)
                def _():
                    issue_in(w + 32 * (i0 + 2), rva, sia)

                wait_in(rvb, sib)

                @pl.when(k > 0)
                def _():
                    wait_out(ub, sob)

                _shuffle_retile(ub, rvb, BLK)
                issue_out(w + 32 * (i0 + 1), ub, sob)

            return carry

        npairs = (nb + 1) // 2
        lax.fori_loop(0, npairs, pair, 0)
        wait_out(ua, soa)

        @pl.when(nb > 1)
        def _():
            wait_out(ub, sob)

    @pl.when(w == 30)
    def _():
        col = NFULL * BLK
        pltpu.async_copy(rows_hbm.at[pl.ds(col // 8, TAIL_A // 8)],
                         rva.at[pl.ds(0, TAIL_A // 8)], sia)
        pltpu.make_async_copy(rows_hbm.at[pl.ds(0, TAIL_A // 8)],
                              rva.at[pl.ds(0, TAIL_A // 8)], sia).wait()
        _shuffle_retile(ua, rva, TAIL_A)
        pltpu.async_copy(ua.at[pl.ds(0, 8), pl.ds(0, TAIL_A)],
                         out_hbm.at[pl.ds(0, 8), pl.ds(col, TAIL_A)], soa)
        pltpu.async_copy(ua.at[pl.ds(8, 8), pl.ds(0, TAIL_A)],
                         out_hbm.at[pl.ds(8, 8), pl.ds(col, TAIL_A)], soa)
        pltpu.make_async_copy(ua.at[pl.ds(0, 8), pl.ds(0, TAIL_A)],
                              out_hbm.at[pl.ds(0, 8), pl.ds(0, TAIL_A)], soa).wait()
        pltpu.make_async_copy(ua.at[pl.ds(8, 8), pl.ds(0, TAIL_A)],
                              out_hbm.at[pl.ds(8, 8), pl.ds(0, TAIL_A)], soa).wait()

    @pl.when(w == 31)
    def _():
        colw = Z_DIM - 64
        col = pl.multiple_of(jnp.int32(Z_DIM - 64), 128)
        pltpu.async_copy(rows_hbm.at[pl.ds(colw // 8, 8)],
                         rva.at[pl.ds(0, 8)], sia)
        pltpu.make_async_copy(rows_hbm.at[pl.ds(0, 8)],
                              rva.at[pl.ds(0, 8)], sia).wait()
        _shuffle_retile(ua, rva, 128)
        pltpu.async_copy(ua.at[pl.ds(0, 8), pl.ds(0, 128)],
                         out_hbm.at[pl.ds(0, 8), pl.ds(col, 128)], soa)
        pltpu.async_copy(ua.at[pl.ds(8, 8), pl.ds(0, 128)],
                         out_hbm.at[pl.ds(8, 8), pl.ds(col, 128)], soa)
        pltpu.make_async_copy(ua.at[pl.ds(0, 8), pl.ds(0, 128)],
                              out_hbm.at[pl.ds(0, 8), pl.ds(0, 128)], soa).wait()
        pltpu.make_async_copy(ua.at[pl.ds(8, 8), pl.ds(0, 128)],
                              out_hbm.at[pl.ds(8, 8), pl.ds(0, 128)], soa).wait()


def _gather_body(z_hbm, p_hbm, out_hbm, ia, ra, ib, rb,
                 siia, sga, sooa, siib, sgb, soob):
    w = _worker_id()
    start = BASE_PER_W * w + jnp.minimum(w, REM)
    n = jnp.where(w < REM, BASE_PER_W + 1, BASE_PER_W)

    def issue_idx(c, iv, sem):
        pltpu.async_copy(p_hbm.at[pl.ds(c * CHUNK, CHUNK)], iv, sem)

    def wait_idx(iv, sem):
        pltpu.make_async_copy(p_hbm.at[pl.ds(0, CHUNK)], iv, sem).wait()

    def issue_g(iv, rv, sem):
        pltpu.async_copy(z_hbm.at[iv], rv, sem)

    def wait_g(iv, rv, sem):
        pltpu.make_async_copy(z_hbm.at[iv], rv, sem).wait()

    def issue_out(c, rv, sem):
        pltpu.async_copy(rv, out_hbm.at[pl.ds(c * CHUNK, CHUNK)], sem)

    def wait_out(rv, sem):
        pltpu.make_async_copy(rv, out_hbm.at[pl.ds(0, CHUNK)], sem).wait()

    issue_idx(start, ia, siia)

    def pair(k, carry):
        i0 = 2 * k
        c0 = start + i0

        @pl.when(i0 + 1 < n)
        def _():
            issue_idx(c0 + 1, ib, siib)

        wait_idx(ia, siia)

        @pl.when(k > 0)
        def _():
            wait_out(ra, sooa)

        issue_g(ia, ra, sga)

        @pl.when(i0 + 1 < n)
        def _():
            wait_idx(ib, siib)

            @pl.when(k > 0)
            def _():
                wait_out(rb, soob)

            issue_g(ib, rb, sgb)

        wait_g(ia, ra, sga)
        issue_out(c0, ra, sooa)

        @pl.when(i0 + 2 < n)
        def _():
            issue_idx(c0 + 2, ia, siia)

        @pl.when(i0 + 1 < n)
        def _():
            wait_g(ib, rb, sgb)
            issue_out(c0 + 1, rb, soob)

        return carry

    lax.fori_loop(0, (n + 1) // 2, pair, 0)
    wait_out(ra, sooa)

    @pl.when(n > 1)
    def _():
        wait_out(rb, soob)


def _mesh():
    return plsc.VectorSubcoreMesh(
        core_axis_name="c", subcore_axis_name="s", num_cores=NC, num_subcores=NS
    )


def _tile_scratch():
    return [
        pltpu.VMEM((8, BLK), jnp.float32),
        pltpu.VMEM((8, BLK), jnp.float32),
        pltpu.VMEM((BLK // 8, 128), jnp.float32),
        pltpu.VMEM((8, BLK), jnp.float32),
        pltpu.VMEM((8, BLK), jnp.float32),
        pltpu.VMEM((BLK // 8, 128), jnp.float32),
        pltpu.SemaphoreType.DMA,
        pltpu.SemaphoreType.DMA,
        pltpu.SemaphoreType.DMA,
        pltpu.SemaphoreType.DMA,
    ]


@jax.jit
def _permute_gather(z, permute):
    zt = z.T

    z_rows128 = pl.kernel(
        _detile_body,
        out_type=jax.ShapeDtypeStruct((Z_DIM // 8, 128), jnp.float32),
        mesh=_mesh(),
        scratch_types=_tile_scratch(),
        compiler_params=pltpu.CompilerParams(use_tc_tiling_on_sc=True, needs_layout_passes=False, disable_bounds_checks=True),
    )(zt)

    out_lin = pl.kernel(
        _gather_body,
        out_type=jax.ShapeDtypeStruct((Z_DIM, FEAT), jnp.float32),
        mesh=_mesh(),
        scratch_types=[
            pltpu.VMEM((CHUNK,), jnp.int32),
            pltpu.VMEM((CHUNK, FEAT), jnp.float32),
            pltpu.VMEM((CHUNK,), jnp.int32),
            pltpu.VMEM((CHUNK, FEAT), jnp.float32),
            pltpu.SemaphoreType.DMA,
            pltpu.SemaphoreType.DMA,
            pltpu.SemaphoreType.DMA,
            pltpu.SemaphoreType.DMA,
            pltpu.SemaphoreType.DMA,
            pltpu.SemaphoreType.DMA,
        ],
        compiler_params=pltpu.CompilerParams(use_tc_tiling_on_sc=False),
    )(z_rows128.reshape(Z_DIM, FEAT), permute)

    out_t = pl.kernel(
        _retile_body,
        out_type=jax.ShapeDtypeStruct((FEAT, Z_DIM), jnp.float32),
        mesh=_mesh(),
        scratch_types=[
            pltpu.VMEM((FEAT, BLK), jnp.float32),
            pltpu.VMEM((BLK // 8, 128), jnp.float32),
            pltpu.VMEM((FEAT, BLK), jnp.float32),
            pltpu.VMEM((BLK // 8, 128), jnp.float32),
            pltpu.SemaphoreType.DMA,
            pltpu.SemaphoreType.DMA,
            pltpu.SemaphoreType.DMA,
            pltpu.SemaphoreType.DMA,
        ],
        compiler_params=pltpu.CompilerParams(use_tc_tiling_on_sc=True, needs_layout_passes=False, disable_bounds_checks=True),
    )(out_lin.reshape(Z_DIM // 8, 128))

    return out_t.T


def kernel(z, permute):
    return _permute_gather(z, permute.astype(jnp.int32))


# speedup vs baseline: 1.1347x; 1.1161x over previous
import functools

import jax
import jax.numpy as jnp
from jax import lax
from jax.experimental import pallas as pl
from jax.experimental.pallas import tpu as pltpu
from jax.experimental.pallas import tpu_sc as plsc

Z_DIM = 1_000_000
FEAT = 16
NC = 2
NS = 16
NW = NC * NS

BLK = 1024
NFULL = Z_DIM // BLK
TAIL = Z_DIM - NFULL * BLK
TAIL_A = 512
TAIL_B = 64
NB_LO = NFULL // NW
NB_REM = NFULL - NB_LO * NW

CHUNK = 2000
NCHUNK = Z_DIM // CHUNK
BASE_PER_W = NCHUNK // NW
REM = NCHUNK - BASE_PER_W * NW


def _worker_id():
    return lax.axis_index("s") * NC + lax.axis_index("c")


def _shuffle_detile(t0, t1, rv, ncols):
    iota = lax.broadcasted_iota(jnp.int32, (16,), 0)

    def body(g, carry):
        c0 = g * 16
        cvec = c0 + iota
        d0 = lax.shift_right_logical(cvec, 3)
        tbase = lax.shift_left(lax.bitwise_and(cvec, 7), 4)
        vals0 = [t0[j, pl.ds(c0, 16)] for j in range(8)]
        vals1 = [t1[j, pl.ds(c0, 16)] for j in range(8)]
        for j in range(8):
            plsc.store_scatter(rv, [d0, tbase + j], vals0[j])
        for j in range(8):
            plsc.store_scatter(rv, [d0, tbase + (j + 8)], vals1[j])
        return carry

    lax.fori_loop(0, ncols // 16, body, 0, unroll=2)


def _shuffle_retile(u, rv, ncols):
    iota = lax.broadcasted_iota(jnp.int32, (16,), 0)
    mjs = [lax.bitwise_and(iota + j, 15) for j in range(16)]

    def body(g, carry):
        c0 = g * 16
        cvec = c0 + iota
        base16 = lax.shift_left(cvec, 4)
        flats = [base16 + mjs[j] for j in range(16)]
        vals = [
            plsc.load_gather(
                rv, [lax.shift_right_logical(f, 7), lax.bitwise_and(f, 127)])
            for f in flats
        ]
        for j in range(16):
            plsc.store_scatter(u, [mjs[j], cvec], vals[j])
        return carry

    lax.fori_loop(0, ncols // 16, body, 0, unroll=2)


def _detile_body(zt_hbm, out_hbm, t0a, t1a, rva, t0b, t1b, rvb,
                 sia, sib, soa, sob):
    w = _worker_id()
    nb = jnp.where(w < NB_REM, NB_LO + 1, NB_LO)

    def issue_in(cb, t0, t1, sem):
        col = cb * BLK
        pltpu.async_copy(zt_hbm.at[pl.ds(0, 8), pl.ds(col, BLK)], t0, sem)
        pltpu.async_copy(zt_hbm.at[pl.ds(8, 8), pl.ds(col, BLK)], t1, sem)

    def wait_in(t0, t1, sem):
        pltpu.make_async_copy(zt_hbm.at[pl.ds(0, 8), pl.ds(0, BLK)], t0, sem).wait()
        pltpu.make_async_copy(zt_hbm.at[pl.ds(8, 8), pl.ds(0, BLK)], t1, sem).wait()

    def issue_out(cb, rv, sem):
        pltpu.async_copy(rv, out_hbm.at[pl.ds(cb * (BLK // 8), BLK // 8)], sem)

    def wait_out(rv, sem):
        pltpu.make_async_copy(rv, out_hbm.at[pl.ds(0, BLK // 8)], sem).wait()

    @pl.when(nb > 0)
    def _():
        issue_in(w, t0a, t1a, sia)

        def pair(k, carry):
            i0 = 2 * k
            cb0 = w + 32 * i0

            @pl.when(i0 + 1 < nb)
            def _():
                issue_in(w + 32 * (i0 + 1), t0b, t1b, sib)

            wait_in(t0a, t1a, sia)
            _shuffle_detile(t0a, t1a, rva, BLK)

            @pl.when(k > 0)
            def _():
                wait_out(rva, soa)

            issue_out(cb0, rva, soa)

            @pl.when(i0 + 1 < nb)
            def _():
                @pl.when(i0 + 2 < nb)
                def _():
                    issue_in(w + 32 * (i0 + 2), t0a, t1a, sia)

                wait_in(t0b, t1b, sib)
                _shuffle_detile(t0b, t1b, rvb, BLK)

                @pl.when(k > 0)
                def _():
                    wait_out(rvb, sob)

                issue_out(w + 32 * (i0 + 1), rvb, sob)

            return carry

        npairs = (nb + 1) // 2
        lax.fori_loop(0, npairs, pair, 0)
        wait_out(rva, soa)

        @pl.when(nb > 1)
        def _():
            wait_out(rvb, sob)

    @pl.when(w == 30)
    def _():
        col = NFULL * BLK
        pltpu.async_copy(zt_hbm.at[pl.ds(0, 8), pl.ds(col, TAIL_A)],
                         t0a.at[:, pl.ds(0, TAIL_A)], sia)
        pltpu.async_copy(zt_hbm.at[pl.ds(8, 8), pl.ds(col, TAIL_A)],
                         t1a.at[:, pl.ds(0, TAIL_A)], sia)
        pltpu.make_async_copy(zt_hbm.at[pl.ds(0, 8), pl.ds(0, TAIL_A)],
                              t0a.at[:, pl.ds(0, TAIL_A)], sia).wait()
        pltpu.make_async_copy(zt_hbm.at[pl.ds(8, 8), pl.ds(0, TAIL_A)],
                              t1a.at[:, pl.ds(0, TAIL_A)], sia).wait()
        _shuffle_detile(t0a, t1a, rva, TAIL_A)
        pltpu.async_copy(rva.at[pl.ds(0, TAIL_A // 8)],
                         out_hbm.at[pl.ds(col // 8, TAIL_A // 8)], soa)
        pltpu.make_async_copy(rva.at[pl.ds(0, TAIL_A // 8)],
                              out_hbm.at[pl.ds(0, TAIL_A // 8)], soa).wait()

    @pl.when(w == 31)
    def _():
        col = pl.multiple_of(jnp.int32(Z_DIM - 64), 128)
        colw = Z_DIM - 64
        pltpu.async_copy(zt_hbm.at[pl.ds(0, 8), pl.ds(col, 128)],
                         t0a.at[:, pl.ds(0, 128)], sia)
        pltpu.async_copy(zt_hbm.at[pl.ds(8, 8), pl.ds(col, 128)],
                         t1a.at[:, pl.ds(0, 128)], sia)
        pltpu.make_async_copy(zt_hbm.at[pl.ds(0, 8), pl.ds(0, 128)],
                              t0a.at[:, pl.ds(0, 128)], sia).wait()
        pltpu.make_async_copy(zt_hbm.at[pl.ds(8, 8), pl.ds(0, 128)],
                              t1a.at[:, pl.ds(0, 128)], sia).wait()
        _shuffle_detile(t0a, t1a, rva, 128)
        pltpu.async_copy(rva.at[pl.ds(0, 8)],
                         out_hbm.at[pl.ds(colw // 8, 8)], soa)
        pltpu.make_async_copy(rva.at[pl.ds(0, 8)],
                              out_hbm.at[pl.ds(0, 8)], soa).wait()


def _retile_body(rows_hbm, out_hbm, ua, rva, ub, rvb,
                 sia, sib, soa, sob):
    w = _worker_id()
    nb = jnp.where(w < NB_REM, NB_LO + 1, NB_LO)

    def issue_in(cb, rv, sem):
        pltpu.async_copy(rows_hbm.at[pl.ds(cb * (BLK // 8), BLK // 8)], rv, sem)

    def wait_in(rv, sem):
        pltpu.make_async_copy(rows_hbm.at[pl.ds(0, BLK // 8)], rv, sem).wait()

    def issue_out(cb, u, sem):
        col = cb * BLK
        pltpu.async_copy(u.at[pl.ds(0, 8)], out_hbm.at[pl.ds(0, 8), pl.ds(col, BLK)], sem)
        pltpu.async_copy(u.at[pl.ds(8, 8)], out_hbm.at[pl.ds(8, 8), pl.ds(col, BLK)], sem)

    def wait_out(u, sem):
        pltpu.make_async_copy(u.at[pl.ds(0, 8)], out_hbm.at[pl.ds(0, 8), pl.ds(0, BLK)], sem).wait()
        pltpu.make_async_copy(u.at[pl.ds(8, 8)], out_hbm.at[pl.ds(8, 8), pl.ds(0, BLK)], sem).wait()

    @pl.when(nb > 0)
    def _():
        issue_in(w, rva, sia)

        def pair(k, carry):
            i0 = 2 * k
            cb0 = w + 32 * i0

            @pl.when(i0 + 1 < nb)
            def _():
                issue_in(w + 32 * (i0 + 1), rvb, sib)

            wait_in(rva, sia)

            @pl.when(k > 0)
            def _():
                wait_out(ua, soa)

            _shuffle_retile(ua, rva, BLK)
            issue_out(cb0, ua, soa)

            @pl.when(i0 + 1 < nb)
            def _():
                @pl.when(i0 + 2 < nb)
                def _():
                    issue_in(w + 32 * (i0 + 2), rva, sia)

                wait_in(rvb, sib)

                @pl.when(k > 0)
                def _():
                    wait_out(ub, sob)

                _shuffle_retile(ub, rvb, BLK)
                issue_out(w + 32 * (i0 + 1), ub, sob)

            return carry

        npairs = (nb + 1) // 2
        lax.fori_loop(0, npairs, pair, 0)
        wait_out(ua, soa)

        @pl.when(nb > 1)
        def _():
            wait_out(ub, sob)

    @pl.when(w == 30)
    def _():
        col = NFULL * BLK
        pltpu.async_copy(rows_hbm.at[pl.ds(col // 8, TAIL_A // 8)],
                         rva.at[pl.ds(0, TAIL_A // 8)], sia)
        pltpu.make_async_copy(rows_hbm.at[pl.ds(0, TAIL_A // 8)],
                              rva.at[pl.ds(0, TAIL_A // 8)], sia).wait()
        _shuffle_retile(ua, rva, TAIL_A)
        pltpu.async_copy(ua.at[pl.ds(0, 8), pl.ds(0, TAIL_A)],
                         out_hbm.at[pl.ds(0, 8), pl.ds(col, TAIL_A)], soa)
        pltpu.async_copy(ua.at[pl.ds(8, 8), pl.ds(0, TAIL_A)],
                         out_hbm.at[pl.ds(8, 8), pl.ds(col, TAIL_A)], soa)
        pltpu.make_async_copy(ua.at[pl.ds(0, 8), pl.ds(0, TAIL_A)],
                              out_hbm.at[pl.ds(0, 8), pl.ds(0, TAIL_A)], soa).wait()
        pltpu.make_async_copy(ua.at[pl.ds(8, 8), pl.ds(0, TAIL_A)],
                              out_hbm.at[pl.ds(8, 8), pl.ds(0, TAIL_A)], soa).wait()

    @pl.when(w == 31)
    def _():
        colw = Z_DIM - 64
        col = pl.multiple_of(jnp.int32(Z_DIM - 64), 128)
        pltpu.async_copy(rows_hbm.at[pl.ds(colw // 8, 8)],
                         rva.at[pl.ds(0, 8)], sia)
        pltpu.make_async_copy(rows_hbm.at[pl.ds(0, 8)],
                              rva.at[pl.ds(0, 8)], sia).wait()
        _shuffle_retile(ua, rva, 128)
        pltpu.async_copy(ua.at[pl.ds(0, 8), pl.ds(0, 128)],
                         out_hbm.at[pl.ds(0, 8), pl.ds(col, 128)], soa)
        pltpu.async_copy(ua.at[pl.ds(8, 8), pl.ds(0, 128)],
                         out_hbm.at[pl.ds(8, 8), pl.ds(col, 128)], soa)
        pltpu.make_async_copy(ua.at[pl.ds(0, 8), pl.ds(0, 128)],
                              out_hbm.at[pl.ds(0, 8), pl.ds(0, 128)], soa).wait()
        pltpu.make_async_copy(ua.at[pl.ds(8, 8), pl.ds(0, 128)],
                              out_hbm.at[pl.ds(8, 8), pl.ds(0, 128)], soa).wait()


def _gather_body(z_hbm, p_hbm, out_hbm, ia, ra, ib, rb,
                 siia, sga, sooa, siib, sgb, soob):
    w = _worker_id()
    start = BASE_PER_W * w + jnp.minimum(w, REM)
    n = jnp.where(w < REM, BASE_PER_W + 1, BASE_PER_W)

    def issue_idx(c, iv, sem):
        pltpu.async_copy(p_hbm.at[pl.ds(c * CHUNK, CHUNK)], iv, sem)

    def wait_idx(iv, sem):
        pltpu.make_async_copy(p_hbm.at[pl.ds(0, CHUNK)], iv, sem).wait()

    def issue_g(iv, rv, sem):
        pltpu.async_copy(z_hbm.at[iv], rv, sem)

    def wait_g(iv, rv, sem):
        pltpu.make_async_copy(z_hbm.at[iv], rv, sem).wait()

    def issue_out(c, rv, sem):
        pltpu.async_copy(rv, out_hbm.at[pl.ds(c * CHUNK, CHUNK)], sem)

    def wait_out(rv, sem):
        pltpu.make_async_copy(rv, out_hbm.at[pl.ds(0, CHUNK)], sem).wait()

    issue_idx(start, ia, siia)

    def pair(k, carry):
        i0 = 2 * k
        c0 = start + i0

        @pl.when(i0 + 1 < n)
        def _():
            issue_idx(c0 + 1, ib, siib)

        wait_idx(ia, siia)

        @pl.when(k > 0)
        def _():
            wait_out(ra, sooa)

        issue_g(ia, ra, sga)

        @pl.when(i0 + 1 < n)
        def _():
            wait_idx(ib, siib)

            @pl.when(k > 0)
            def _():
                wait_out(rb, soob)

            issue_g(ib, rb, sgb)

        wait_g(ia, ra, sga)
        issue_out(c0, ra, sooa)

        @pl.when(i0 + 2 < n)
        def _():
            issue_idx(c0 + 2, ia, siia)

        @pl.when(i0 + 1 < n)
        def _():
            wait_g(ib, rb, sgb)
            issue_out(c0 + 1, rb, soob)

        return carry

    lax.fori_loop(0, (n + 1) // 2, pair, 0)
    wait_out(ra, sooa)

    @pl.when(n > 1)
    def _():
        wait_out(rb, soob)


def _mesh():
    return plsc.VectorSubcoreMesh(
        core_axis_name="c", subcore_axis_name="s", num_cores=NC, num_subcores=NS
    )


def _tile_scratch():
    return [
        pltpu.VMEM((8, BLK), jnp.float32),
        pltpu.VMEM((8, BLK), jnp.float32),
        pltpu.VMEM((BLK // 8, 128), jnp.float32),
        pltpu.VMEM((8, BLK), jnp.float32),
        pltpu.VMEM((8, BLK), jnp.float32),
        pltpu.VMEM((BLK // 8, 128), jnp.float32),
        pltpu.SemaphoreType.DMA,
        pltpu.SemaphoreType.DMA,
        pltpu.SemaphoreType.DMA,
        pltpu.SemaphoreType.DMA,
    ]


@jax.jit
def _permute_gather(z, permute):
    zt = z.T

    z_rows128 = pl.kernel(
        _detile_body,
        out_type=jax.ShapeDtypeStruct((Z_DIM // 8, 128), jnp.float32),
        mesh=_mesh(),
        scratch_types=_tile_scratch(),
        compiler_params=pltpu.CompilerParams(use_tc_tiling_on_sc=True, needs_layout_passes=False, disable_bounds_checks=True),
    )(zt)

    out_lin = pl.kernel(
        _gather_body,
        out_type=jax.ShapeDtypeStruct((Z_DIM, FEAT), jnp.float32),
        mesh=_mesh(),
        scratch_types=[
            pltpu.VMEM((CHUNK,), jnp.int32),
            pltpu.VMEM((CHUNK, FEAT), jnp.float32),
            pltpu.VMEM((CHUNK,), jnp.int32),
            pltpu.VMEM((CHUNK, FEAT), jnp.float32),
            pltpu.SemaphoreType.DMA,
            pltpu.SemaphoreType.DMA,
            pltpu.SemaphoreType.DMA,
            pltpu.SemaphoreType.DMA,
            pltpu.SemaphoreType.DMA,
            pltpu.SemaphoreType.DMA,
        ],
        compiler_params=pltpu.CompilerParams(use_tc_tiling_on_sc=False),
    )(z_rows128.reshape(Z_DIM, FEAT), permute)

    out_t = pl.kernel(
        _retile_body,
        out_type=jax.ShapeDtypeStruct((FEAT, Z_DIM), jnp.float32),
        mesh=_mesh(),
        scratch_types=[
            pltpu.VMEM((FEAT, BLK), jnp.float32),
            pltpu.VMEM((BLK // 8, 128), jnp.float32),
            pltpu.VMEM((FEAT, BLK), jnp.float32),
            pltpu.VMEM((BLK // 8, 128), jnp.float32),
            pltpu.SemaphoreType.DMA,
            pltpu.SemaphoreType.DMA,
            pltpu.SemaphoreType.DMA,
            pltpu.SemaphoreType.DMA,
        ],
        compiler_params=pltpu.CompilerParams(use_tc_tiling_on_sc=True, needs_layout_passes=False, disable_bounds_checks=True),
    )(out_lin.reshape(Z_DIM // 8, 128))

    return out_t.T


def kernel(z, permute):
    return _permute_gather(z, permute.astype(jnp.int32))


# speedup vs baseline: 1.1355x; 1.0007x over previous
import functools

import jax
import jax.numpy as jnp
from jax import lax
from jax.experimental import pallas as pl
from jax.experimental.pallas import tpu as pltpu
from jax.experimental.pallas import tpu_sc as plsc

Z_DIM = 1_000_000
FEAT = 16
NC = 2
NS = 16
NW = NC * NS

BLK = 1024
NFULL = Z_DIM // BLK
TAIL = Z_DIM - NFULL * BLK
TAIL_A = 512
TAIL_B = 64
NB_LO = NFULL // NW
NB_REM = NFULL - NB_LO * NW

CHUNK = 2000
NCHUNK = Z_DIM // CHUNK
BASE_PER_W = NCHUNK // NW
REM = NCHUNK - BASE_PER_W * NW


def _worker_id():
    return lax.axis_index("s") * NC + lax.axis_index("c")


def _shuffle_detile(t01, rv, ncols):
    iota = lax.broadcasted_iota(jnp.int32, (16,), 0)
    mjs = [lax.bitwise_and(iota + j, 15) for j in range(16)]

    def body(g, carry):
        c0 = g * 16
        cvec = c0 + iota
        base16 = lax.shift_left(cvec, 4)
        vals = [plsc.load_gather(t01, [mjs[j], cvec]) for j in range(16)]
        for j in range(16):
            f = base16 + mjs[j]
            plsc.store_scatter(
                rv, [lax.shift_right_logical(f, 7), lax.bitwise_and(f, 127)],
                vals[j])
        return carry

    lax.fori_loop(0, ncols // 16, body, 0, unroll=2)


def _shuffle_retile(u, rv, ncols):
    iota = lax.broadcasted_iota(jnp.int32, (16,), 0)
    mjs = [lax.bitwise_and(iota + j, 15) for j in range(16)]

    def body(g, carry):
        c0 = g * 16
        cvec = c0 + iota
        base16 = lax.shift_left(cvec, 4)
        flats = [base16 + mjs[j] for j in range(16)]
        vals = [
            plsc.load_gather(
                rv, [lax.shift_right_logical(f, 7), lax.bitwise_and(f, 127)])
            for f in flats
        ]
        for j in range(16):
            plsc.store_scatter(u, [mjs[j], cvec], vals[j])
        return carry

    lax.fori_loop(0, ncols // 16, body, 0, unroll=2)


def _detile_body(zt_hbm, out_hbm, t01a, rva, t01b, rvb,
                 sia, sib, soa, sob):
    w = _worker_id()
    nb = jnp.where(w < NB_REM, NB_LO + 1, NB_LO)

    def issue_in(cb, t01, sem):
        col = cb * BLK
        pltpu.async_copy(zt_hbm.at[pl.ds(0, 8), pl.ds(col, BLK)],
                         t01.at[pl.ds(0, 8)], sem)
        pltpu.async_copy(zt_hbm.at[pl.ds(8, 8), pl.ds(col, BLK)],
                         t01.at[pl.ds(8, 8)], sem)

    def wait_in(t01, sem):
        pltpu.make_async_copy(zt_hbm.at[pl.ds(0, 8), pl.ds(0, BLK)],
                              t01.at[pl.ds(0, 8)], sem).wait()
        pltpu.make_async_copy(zt_hbm.at[pl.ds(8, 8), pl.ds(0, BLK)],
                              t01.at[pl.ds(8, 8)], sem).wait()

    def issue_out(cb, rv, sem):
        pltpu.async_copy(rv, out_hbm.at[pl.ds(cb * (BLK // 8), BLK // 8)], sem)

    def wait_out(rv, sem):
        pltpu.make_async_copy(rv, out_hbm.at[pl.ds(0, BLK // 8)], sem).wait()

    @pl.when(nb > 0)
    def _():
        issue_in(w, t01a, sia)

        def pair(k, carry):
            i0 = 2 * k
            cb0 = w + 32 * i0

            @pl.when(i0 + 1 < nb)
            def _():
                issue_in(w + 32 * (i0 + 1), t01b, sib)

            wait_in(t01a, sia)
            _shuffle_detile(t01a, rva, BLK)

            @pl.when(k > 0)
            def _():
                wait_out(rva, soa)

            issue_out(cb0, rva, soa)

            @pl.when(i0 + 1 < nb)
            def _():
                @pl.when(i0 + 2 < nb)
                def _():
                    issue_in(w + 32 * (i0 + 2), t01a, sia)

                wait_in(t01b, sib)
                _shuffle_detile(t01b, rvb, BLK)

                @pl.when(k > 0)
                def _():
                    wait_out(rvb, sob)

                issue_out(w + 32 * (i0 + 1), rvb, sob)

            return carry

        npairs = (nb + 1) // 2
        lax.fori_loop(0, npairs, pair, 0)
        wait_out(rva, soa)

        @pl.when(nb > 1)
        def _():
            wait_out(rvb, sob)

    @pl.when(w == 30)
    def _():
        col = NFULL * BLK
        pltpu.async_copy(zt_hbm.at[pl.ds(0, 8), pl.ds(col, TAIL_A)],
                         t01a.at[pl.ds(0, 8), pl.ds(0, TAIL_A)], sia)
        pltpu.async_copy(zt_hbm.at[pl.ds(8, 8), pl.ds(col, TAIL_A)],
                         t01a.at[pl.ds(8, 8), pl.ds(0, TAIL_A)], sia)
        pltpu.make_async_copy(zt_hbm.at[pl.ds(0, 8), pl.ds(0, TAIL_A)],
                              t01a.at[pl.ds(0, 8), pl.ds(0, TAIL_A)], sia).wait()
        pltpu.make_async_copy(zt_hbm.at[pl.ds(8, 8), pl.ds(0, TAIL_A)],
                              t01a.at[pl.ds(8, 8), pl.ds(0, TAIL_A)], sia).wait()
        _shuffle_detile(t01a, rva, TAIL_A)
        pltpu.async_copy(rva.at[pl.ds(0, TAIL_A // 8)],
                         out_hbm.at[pl.ds(col // 8, TAIL_A // 8)], soa)
        pltpu.make_async_copy(rva.at[pl.ds(0, TAIL_A // 8)],
                              out_hbm.at[pl.ds(0, TAIL_A // 8)], soa).wait()

    @pl.when(w == 31)
    def _():
        col = pl.multiple_of(jnp.int32(Z_DIM - 64), 128)
        colw = Z_DIM - 64
        pltpu.async_copy(zt_hbm.at[pl.ds(0, 8), pl.ds(col, 128)],
                         t01a.at[pl.ds(0, 8), pl.ds(0, 128)], sia)
        pltpu.async_copy(zt_hbm.at[pl.ds(8, 8), pl.ds(col, 128)],
                         t01a.at[pl.ds(8, 8), pl.ds(0, 128)], sia)
        pltpu.make_async_copy(zt_hbm.at[pl.ds(0, 8), pl.ds(0, 128)],
                              t01a.at[pl.ds(0, 8), pl.ds(0, 128)], sia).wait()
        pltpu.make_async_copy(zt_hbm.at[pl.ds(8, 8), pl.ds(0, 128)],
                              t01a.at[pl.ds(8, 8), pl.ds(0, 128)], sia).wait()
        _shuffle_detile(t01a, rva, 128)
        pltpu.async_copy(rva.at[pl.ds(0, 8)],
                         out_hbm.at[pl.ds(colw // 8, 8)], soa)
        pltpu.make_async_copy(rva.at[pl.ds(0, 8)],
                              out_hbm.at[pl.ds(0, 8)], soa).wait()


def _retile_body(rows_hbm, out_hbm, ua, rva, ub, rvb,
                 sia, sib, soa, sob):
    w = _worker_id()
    nb = jnp.where(w < NB_REM, NB_LO + 1, NB_LO)

    def issue_in(cb, rv, sem):
        pltpu.async_copy(rows_hbm.at[pl.ds(cb * (BLK // 8), BLK // 8)], rv, sem)

    def wait_in(rv, sem):
        pltpu.make_async_copy(rows_hbm.at[pl.ds(0, BLK // 8)], rv, sem).wait()

    def issue_out(cb, u, sem):
        col = cb * BLK
        pltpu.async_copy(u.at[pl.ds(0, 8)], out_hbm.at[pl.ds(0, 8), pl.ds(col, BLK)], sem)
        pltpu.async_copy(u.at[pl.ds(8, 8)], out_hbm.at[pl.ds(8, 8), pl.ds(col, BLK)], sem)

    def wait_out(u, sem):
        pltpu.make_async_copy(u.at[pl.ds(0, 8)], out_hbm.at[pl.ds(0, 8), pl.ds(0, BLK)], sem).wait()
        pltpu.make_async_copy(u.at[pl.ds(8, 8)], out_hbm.at[pl.ds(8, 8), pl.ds(0, BLK)], sem).wait()

    @pl.when(nb > 0)
    def _():
        issue_in(w, rva, sia)

        def pair(k, carry):
            i0 = 2 * k
            cb0 = w + 32 * i0

            @pl.when(i0 + 1 < nb)
            def _():
                issue_in(w + 32 * (i0 + 1), rvb, sib)

            wait_in(rva, sia)

            @pl.when(k > 0)
            def _():
                wait_out(ua, soa)

            _shuffle_retile(ua, rva, BLK)
            issue_out(cb0, ua, soa)

            @pl.when(i0 + 1 < nb)
            def _():
                @pl.when(i0 + 2 < nb)
                def _():
                    issue_in(w + 32 * (i0 + 2), rva, sia)

                wait_in(rvb, sib)

                @pl.when(k > 0)
                def _():
                    wait_out(ub, sob)

                _shuffle_retile(ub, rvb, BLK)
                issue_out(w + 32 * (i0 + 1), ub, sob)

            return carry

        npairs = (nb + 1) // 2
        lax.fori_loop(0, npairs, pair, 0)
        wait_out(ua, soa)

        @pl.when(nb > 1)
        def _():
            wait_out(ub, sob)

    @pl.when(w == 30)
    def _():
        col = NFULL * BLK
        pltpu.async_copy(rows_hbm.at[pl.ds(col // 8, TAIL_A // 8)],
                         rva.at[pl.ds(0, TAIL_A // 8)], sia)
        pltpu.make_async_copy(rows_hbm.at[pl.ds(0, TAIL_A // 8)],
                              rva.at[pl.ds(0, TAIL_A // 8)], sia).wait()
        _shuffle_retile(ua, rva, TAIL_A)
        pltpu.async_copy(ua.at[pl.ds(0, 8), pl.ds(0, TAIL_A)],
                         out_hbm.at[pl.ds(0, 8), pl.ds(col, TAIL_A)], soa)
        pltpu.async_copy(ua.at[pl.ds(8, 8), pl.ds(0, TAIL_A)],
                         out_hbm.at[pl.ds(8, 8), pl.ds(col, TAIL_A)], soa)
        pltpu.make_async_copy(ua.at[pl.ds(0, 8), pl.ds(0, TAIL_A)],
                              out_hbm.at[pl.ds(0, 8), pl.ds(0, TAIL_A)], soa).wait()
        pltpu.make_async_copy(ua.at[pl.ds(8, 8), pl.ds(0, TAIL_A)],
                              out_hbm.at[pl.ds(8, 8), pl.ds(0, TAIL_A)], soa).wait()

    @pl.when(w == 31)
    def _():
        colw = Z_DIM - 64
        col = pl.multiple_of(jnp.int32(Z_DIM - 64), 128)
        pltpu.async_copy(rows_hbm.at[pl.ds(colw // 8, 8)],
                         rva.at[pl.ds(0, 8)], sia)
        pltpu.make_async_copy(rows_hbm.at[pl.ds(0, 8)],
                              rva.at[pl.ds(0, 8)], sia).wait()
        _shuffle_retile(ua, rva, 128)
        pltpu.async_copy(ua.at[pl.ds(0, 8), pl.ds(0, 128)],
                         out_hbm.at[pl.ds(0, 8), pl.ds(col, 128)], soa)
        pltpu.async_copy(ua.at[pl.ds(8, 8), pl.ds(0, 128)],
                         out_hbm.at[pl.ds(8, 8), pl.ds(col, 128)], soa)
        pltpu.make_async_copy(ua.at[pl.ds(0, 8), pl.ds(0, 128)],
                              out_hbm.at[pl.ds(0, 8), pl.ds(0, 128)], soa).wait()
        pltpu.make_async_copy(ua.at[pl.ds(8, 8), pl.ds(0, 128)],
                              out_hbm.at[pl.ds(8, 8), pl.ds(0, 128)], soa).wait()


def _gather_body(z_hbm, p_hbm, out_hbm, ia, ra, ib, rb,
                 siia, sga, sooa, siib, sgb, soob):
    w = _worker_id()
    start = BASE_PER_W * w + jnp.minimum(w, REM)
    n = jnp.where(w < REM, BASE_PER_W + 1, BASE_PER_W)

    def issue_idx(c, iv, sem):
        pltpu.async_copy(p_hbm.at[pl.ds(c * CHUNK, CHUNK)], iv, sem)

    def wait_idx(iv, sem):
        pltpu.make_async_copy(p_hbm.at[pl.ds(0, CHUNK)], iv, sem).wait()

    def issue_g(iv, rv, sem):
        pltpu.async_copy(z_hbm.at[iv], rv, sem)

    def wait_g(iv, rv, sem):
        pltpu.make_async_copy(z_hbm.at[iv], rv, sem).wait()

    def issue_out(c, rv, sem):
        pltpu.async_copy(rv, out_hbm.at[pl.ds(c * CHUNK, CHUNK)], sem)

    def wait_out(rv, sem):
        pltpu.make_async_copy(rv, out_hbm.at[pl.ds(0, CHUNK)], sem).wait()

    issue_idx(start, ia, siia)

    def pair(k, carry):
        i0 = 2 * k
        c0 = start + i0

        @pl.when(i0 + 1 < n)
        def _():
            issue_idx(c0 + 1, ib, siib)

        wait_idx(ia, siia)

        @pl.when(k > 0)
        def _():
            wait_out(ra, sooa)

        issue_g(ia, ra, sga)

        @pl.when(i0 + 1 < n)
        def _():
            wait_idx(ib, siib)

            @pl.when(k > 0)
            def _():
                wait_out(rb, soob)

            issue_g(ib, rb, sgb)

        wait_g(ia, ra, sga)
        issue_out(c0, ra, sooa)

        @pl.when(i0 + 2 < n)
        def _():
            issue_idx(c0 + 2, ia, siia)

        @pl.when(i0 + 1 < n)
        def _():
            wait_g(ib, rb, sgb)
            issue_out(c0 + 1, rb, soob)

        return carry

    lax.fori_loop(0, (n + 1) // 2, pair, 0)
    wait_out(ra, sooa)

    @pl.when(n > 1)
    def _():
        wait_out(rb, soob)


def _mesh():
    return plsc.VectorSubcoreMesh(
        core_axis_name="c", subcore_axis_name="s", num_cores=NC, num_subcores=NS
    )


def _tile_scratch():
    return [
        pltpu.VMEM((FEAT, BLK), jnp.float32),
        pltpu.VMEM((BLK // 8, 128), jnp.float32),
        pltpu.VMEM((FEAT, BLK), jnp.float32),
        pltpu.VMEM((BLK // 8, 128), jnp.float32),
        pltpu.SemaphoreType.DMA,
        pltpu.SemaphoreType.DMA,
        pltpu.SemaphoreType.DMA,
        pltpu.SemaphoreType.DMA,
    ]


@jax.jit
def _permute_gather(z, permute):
    zt = z.T

    z_rows128 = pl.kernel(
        _detile_body,
        out_type=jax.ShapeDtypeStruct((Z_DIM // 8, 128), jnp.float32),
        mesh=_mesh(),
        scratch_types=_tile_scratch(),
        compiler_params=pltpu.CompilerParams(use_tc_tiling_on_sc=True, needs_layout_passes=False, disable_bounds_checks=True),
    )(zt)

    out_lin = pl.kernel(
        _gather_body,
        out_type=jax.ShapeDtypeStruct((Z_DIM, FEAT), jnp.float32),
        mesh=_mesh(),
        scratch_types=[
            pltpu.VMEM((CHUNK,), jnp.int32),
            pltpu.VMEM((CHUNK, FEAT), jnp.float32),
            pltpu.VMEM((CHUNK,), jnp.int32),
            pltpu.VMEM((CHUNK, FEAT), jnp.float32),
            pltpu.SemaphoreType.DMA,
            pltpu.SemaphoreType.DMA,
            pltpu.SemaphoreType.DMA,
            pltpu.SemaphoreType.DMA,
            pltpu.SemaphoreType.DMA,
            pltpu.SemaphoreType.DMA,
        ],
        compiler_params=pltpu.CompilerParams(use_tc_tiling_on_sc=False),
    )(z_rows128.reshape(Z_DIM, FEAT), permute)

    out_t = pl.kernel(
        _retile_body,
        out_type=jax.ShapeDtypeStruct((FEAT, Z_DIM), jnp.float32),
        mesh=_mesh(),
        scratch_types=[
            pltpu.VMEM((FEAT, BLK), jnp.float32),
            pltpu.VMEM((BLK // 8, 128), jnp.float32),
            pltpu.VMEM((FEAT, BLK), jnp.float32),
            pltpu.VMEM((BLK // 8, 128), jnp.float32),
            pltpu.SemaphoreType.DMA,
            pltpu.SemaphoreType.DMA,
            pltpu.SemaphoreType.DMA,
            pltpu.SemaphoreType.DMA,
        ],
        compiler_params=pltpu.CompilerParams(use_tc_tiling_on_sc=True, needs_layout_passes=False, disable_bounds_checks=True),
    )(out_lin.reshape(Z_DIM // 8, 128))

    return out_t.T


def kernel(z, permute):
    return _permute_gather(z, permute.astype(jnp.int32))


# speedup vs baseline: 1.1480x; 1.0110x over previous
import functools

import jax
import jax.numpy as jnp
from jax import lax
from jax.experimental import pallas as pl
from jax.experimental.pallas import tpu as pltpu
from jax.experimental.pallas import tpu_sc as plsc

Z_DIM = 1_000_000
FEAT = 16
NC = 2
NS = 16
NW = NC * NS

BLK = 1024
NFULL = Z_DIM // BLK
TAIL = Z_DIM - NFULL * BLK
TAIL_A = 512
TAIL_B = 64
NB_LO = NFULL // NW
NB_REM = NFULL - NB_LO * NW

CHUNK = 2000
NCHUNK = Z_DIM // CHUNK
BASE_PER_W = NCHUNK // NW
REM = NCHUNK - BASE_PER_W * NW


def _worker_id():
    return lax.axis_index("s") * NC + lax.axis_index("c")


def _shuffle_detile(t01, rv, ncols):
    iota = lax.broadcasted_iota(jnp.int32, (16,), 0)
    mjs = [lax.bitwise_and(iota + j, 15) for j in range(16)]

    def body(g, carry):
        c0 = g * 16
        cvec = c0 + iota
        base16 = lax.shift_left(cvec, 4)
        vals = [plsc.load_gather(t01, [mjs[j], cvec]) for j in range(16)]
        for j in range(16):
            f = base16 + mjs[j]
            plsc.store_scatter(
                rv, [lax.shift_right_logical(f, 7), lax.bitwise_and(f, 127)],
                vals[j])
        return carry

    lax.fori_loop(0, ncols // 16, body, 0, unroll=4)


def _shuffle_retile(u, rv, ncols):
    iota = lax.broadcasted_iota(jnp.int32, (16,), 0)
    mjs = [lax.bitwise_and(iota + j, 15) for j in range(16)]

    def body(g, carry):
        c0 = g * 16
        cvec = c0 + iota
        base16 = lax.shift_left(cvec, 4)
        flats = [base16 + mjs[j] for j in range(16)]
        vals = [
            plsc.load_gather(
                rv, [lax.shift_right_logical(f, 7), lax.bitwise_and(f, 127)])
            for f in flats
        ]
        for j in range(16):
            plsc.store_scatter(u, [mjs[j], cvec], vals[j])
        return carry

    lax.fori_loop(0, ncols // 16, body, 0, unroll=4)


def _detile_body(zt_hbm, out_hbm, t01a, rva, t01b, rvb,
                 sia, sib, soa, sob):
    w = _worker_id()
    nb = jnp.where(w < NB_REM, NB_LO + 1, NB_LO)

    def issue_in(cb, t01, sem):
        col = cb * BLK
        pltpu.async_copy(zt_hbm.at[pl.ds(0, 8), pl.ds(col, BLK)],
                         t01.at[pl.ds(0, 8)], sem)
        pltpu.async_copy(zt_hbm.at[pl.ds(8, 8), pl.ds(col, BLK)],
                         t01.at[pl.ds(8, 8)], sem)

    def wait_in(t01, sem):
        pltpu.make_async_copy(zt_hbm.at[pl.ds(0, 8), pl.ds(0, BLK)],
                              t01.at[pl.ds(0, 8)], sem).wait()
        pltpu.make_async_copy(zt_hbm.at[pl.ds(8, 8), pl.ds(0, BLK)],
                              t01.at[pl.ds(8, 8)], sem).wait()

    def issue_out(cb, rv, sem):
        pltpu.async_copy(rv, out_hbm.at[pl.ds(cb * (BLK // 8), BLK // 8)], sem)

    def wait_out(rv, sem):
        pltpu.make_async_copy(rv, out_hbm.at[pl.ds(0, BLK // 8)], sem).wait()

    @pl.when(nb > 0)
    def _():
        issue_in(w, t01a, sia)

        def pair(k, carry):
            i0 = 2 * k
            cb0 = w + 32 * i0

            @pl.when(i0 + 1 < nb)
            def _():
                issue_in(w + 32 * (i0 + 1), t01b, sib)

            wait_in(t01a, sia)
            _shuffle_detile(t01a, rva, BLK)

            @pl.when(k > 0)
            def _():
                wait_out(rva, soa)

            issue_out(cb0, rva, soa)

            @pl.when(i0 + 1 < nb)
            def _():
                @pl.when(i0 + 2 < nb)
                def _():
                    issue_in(w + 32 * (i0 + 2), t01a, sia)

                wait_in(t01b, sib)
                _shuffle_detile(t01b, rvb, BLK)

                @pl.when(k > 0)
                def _():
                    wait_out(rvb, sob)

                issue_out(w + 32 * (i0 + 1), rvb, sob)

            return carry

        npairs = (nb + 1) // 2
        lax.fori_loop(0, npairs, pair, 0)
        wait_out(rva, soa)

        @pl.when(nb > 1)
        def _():
            wait_out(rvb, sob)

    @pl.when(w == 30)
    def _():
        col = NFULL * BLK
        pltpu.async_copy(zt_hbm.at[pl.ds(0, 8), pl.ds(col, TAIL_A)],
                         t01a.at[pl.ds(0, 8), pl.ds(0, TAIL_A)], sia)
        pltpu.async_copy(zt_hbm.at[pl.ds(8, 8), pl.ds(col, TAIL_A)],
                         t01a.at[pl.ds(8, 8), pl.ds(0, TAIL_A)], sia)
        pltpu.make_async_copy(zt_hbm.at[pl.ds(0, 8), pl.ds(0, TAIL_A)],
                              t01a.at[pl.ds(0, 8), pl.ds(0, TAIL_A)], sia).wait()
        pltpu.make_async_copy(zt_hbm.at[pl.ds(8, 8), pl.ds(0, TAIL_A)],
                              t01a.at[pl.ds(8, 8), pl.ds(0, TAIL_A)], sia).wait()
        _shuffle_detile(t01a, rva, TAIL_A)
        pltpu.async_copy(rva.at[pl.ds(0, TAIL_A // 8)],
                         out_hbm.at[pl.ds(col // 8, TAIL_A // 8)], soa)
        pltpu.make_async_copy(rva.at[pl.ds(0, TAIL_A // 8)],
                              out_hbm.at[pl.ds(0, TAIL_A // 8)], soa).wait()

    @pl.when(w == 31)
    def _():
        col = pl.multiple_of(jnp.int32(Z_DIM - 64), 128)
        colw = Z_DIM - 64
        pltpu.async_copy(zt_hbm.at[pl.ds(0, 8), pl.ds(col, 128)],
                         t01a.at[pl.ds(0, 8), pl.ds(0, 128)], sia)
        pltpu.async_copy(zt_hbm.at[pl.ds(8, 8), pl.ds(col, 128)],
                         t01a.at[pl.ds(8, 8), pl.ds(0, 128)], sia)
        pltpu.make_async_copy(zt_hbm.at[pl.ds(0, 8), pl.ds(0, 128)],
                              t01a.at[pl.ds(0, 8), pl.ds(0, 128)], sia).wait()
        pltpu.make_async_copy(zt_hbm.at[pl.ds(8, 8), pl.ds(0, 128)],
                              t01a.at[pl.ds(8, 8), pl.ds(0, 128)], sia).wait()
        _shuffle_detile(t01a, rva, 128)
        pltpu.async_copy(rva.at[pl.ds(0, 8)],
                         out_hbm.at[pl.ds(colw // 8, 8)], soa)
        pltpu.make_async_copy(rva.at[pl.ds(0, 8)],
                              out_hbm.at[pl.ds(0, 8)], soa).wait()


def _retile_body(rows_hbm, out_hbm, ua, rva, ub, rvb,
                 sia, sib, soa, sob):
    w = _worker_id()
    nb = jnp.where(w < NB_REM, NB_LO + 1, NB_LO)

    def issue_in(cb, rv, sem):
        pltpu.async_copy(rows_hbm.at[pl.ds(cb * (BLK // 8), BLK // 8)], rv, sem)

    def wait_in(rv, sem):
        pltpu.make_async_copy(rows_hbm.at[pl.ds(0, BLK // 8)], rv, sem).wait()

    def issue_out(cb, u, sem):
        col = cb * BLK
        pltpu.async_copy(u.at[pl.ds(0, 8)], out_hbm.at[pl.ds(0, 8), pl.ds(col, BLK)], sem)
        pltpu.async_copy(u.at[pl.ds(8, 8)], out_hbm.at[pl.ds(8, 8), pl.ds(col, BLK)], sem)

    def wait_out(u, sem):
        pltpu.make_async_copy(u.at[pl.ds(0, 8)], out_hbm.at[pl.ds(0, 8), pl.ds(0, BLK)], sem).wait()
        pltpu.make_async_copy(u.at[pl.ds(8, 8)], out_hbm.at[pl.ds(8, 8), pl.ds(0, BLK)], sem).wait()

    @pl.when(nb > 0)
    def _():
        issue_in(w, rva, sia)

        def pair(k, carry):
            i0 = 2 * k
            cb0 = w + 32 * i0

            @pl.when(i0 + 1 < nb)
            def _():
                issue_in(w + 32 * (i0 + 1), rvb, sib)

            wait_in(rva, sia)

            @pl.when(k > 0)
            def _():
                wait_out(ua, soa)

            _shuffle_retile(ua, rva, BLK)
            issue_out(cb0, ua, soa)

            @pl.when(i0 + 1 < nb)
            def _():
                @pl.when(i0 + 2 < nb)
                def _():
                    issue_in(w + 32 * (i0 + 2), rva, sia)

                wait_in(rvb, sib)

                @pl.when(k > 0)
                def _():
                    wait_out(ub, sob)

                _shuffle_retile(ub, rvb, BLK)
                issue_out(w + 32 * (i0 + 1), ub, sob)

            return carry

        npairs = (nb + 1) // 2
        lax.fori_loop(0, npairs, pair, 0)
        wait_out(ua, soa)

        @pl.when(nb > 1)
        def _():
            wait_out(ub, sob)

    @pl.when(w == 30)
    def _():
        col = NFULL * BLK
        pltpu.async_copy(rows_hbm.at[pl.ds(col // 8, TAIL_A // 8)],
                         rva.at[pl.ds(0, TAIL_A // 8)], sia)
        pltpu.make_async_copy(rows_hbm.at[pl.ds(0, TAIL_A // 8)],
                              rva.at[pl.ds(0, TAIL_A // 8)], sia).wait()
        _shuffle_retile(ua, rva, TAIL_A)
        pltpu.async_copy(ua.at[pl.ds(0, 8), pl.ds(0, TAIL_A)],
                         out_hbm.at[pl.ds(0, 8), pl.ds(col, TAIL_A)], soa)
        pltpu.async_copy(ua.at[pl.ds(8, 8), pl.ds(0, TAIL_A)],
                         out_hbm.at[pl.ds(8, 8), pl.ds(col, TAIL_A)], soa)
        pltpu.make_async_copy(ua.at[pl.ds(0, 8), pl.ds(0, TAIL_A)],
                              out_hbm.at[pl.ds(0, 8), pl.ds(0, TAIL_A)], soa).wait()
        pltpu.make_async_copy(ua.at[pl.ds(8, 8), pl.ds(0, TAIL_A)],
                              out_hbm.at[pl.ds(8, 8), pl.ds(0, TAIL_A)], soa).wait()

    @pl.when(w == 31)
    def _():
        colw = Z_DIM - 64
        col = pl.multiple_of(jnp.int32(Z_DIM - 64), 128)
        pltpu.async_copy(rows_hbm.at[pl.ds(colw // 8, 8)],
                         rva.at[pl.ds(0, 8)], sia)
        pltpu.make_async_copy(rows_hbm.at[pl.ds(0, 8)],
                              rva.at[pl.ds(0, 8)], sia).wait()
        _shuffle_retile(ua, rva, 128)
        pltpu.async_copy(ua.at[pl.ds(0, 8), pl.ds(0, 128)],
                         out_hbm.at[pl.ds(0, 8), pl.ds(col, 128)], soa)
        pltpu.async_copy(ua.at[pl.ds(8, 8), pl.ds(0, 128)],
                         out_hbm.at[pl.ds(8, 8), pl.ds(col, 128)], soa)
        pltpu.make_async_copy(ua.at[pl.ds(0, 8), pl.ds(0, 128)],
                              out_hbm.at[pl.ds(0, 8), pl.ds(0, 128)], soa).wait()
        pltpu.make_async_copy(ua.at[pl.ds(8, 8), pl.ds(0, 128)],
                              out_hbm.at[pl.ds(8, 8), pl.ds(0, 128)], soa).wait()


def _gather_body(z_hbm, p_hbm, out_hbm, ia, ra, ib, rb,
                 siia, sga, sooa, siib, sgb, soob):
    w = _worker_id()
    start = BASE_PER_W * w + jnp.minimum(w, REM)
    n = jnp.where(w < REM, BASE_PER_W + 1, BASE_PER_W)

    def issue_idx(c, iv, sem):
        pltpu.async_copy(p_hbm.at[pl.ds(c * CHUNK, CHUNK)], iv, sem)

    def wait_idx(iv, sem):
        pltpu.make_async_copy(p_hbm.at[pl.ds(0, CHUNK)], iv, sem).wait()

    def issue_g(iv, rv, sem):
        pltpu.async_copy(z_hbm.at[iv], rv, sem)

    def wait_g(iv, rv, sem):
        pltpu.make_async_copy(z_hbm.at[iv], rv, sem).wait()

    def issue_out(c, rv, sem):
        pltpu.async_copy(rv, out_hbm.at[pl.ds(c * CHUNK, CHUNK)], sem)

    def wait_out(rv, sem):
        pltpu.make_async_copy(rv, out_hbm.at[pl.ds(0, CHUNK)], sem).wait()

    issue_idx(start, ia, siia)

    def pair(k, carry):
        i0 = 2 * k
        c0 = start + i0

        @pl.when(i0 + 1 < n)
        def _():
            issue_idx(c0 + 1, ib, siib)

        wait_idx(ia, siia)

        @pl.when(k > 0)
        def _():
            wait_out(ra, sooa)

        issue_g(ia, ra, sga)

        @pl.when(i0 + 1 < n)
        def _():
            wait_idx(ib, siib)

            @pl.when(k > 0)
            def _():
                wait_out(rb, soob)

            issue_g(ib, rb, sgb)

        wait_g(ia, ra, sga)
        issue_out(c0, ra, sooa)

        @pl.when(i0 + 2 < n)
        def _():
            issue_idx(c0 + 2, ia, siia)

        @pl.when(i0 + 1 < n)
        def _():
            wait_g(ib, rb, sgb)
            issue_out(c0 + 1, rb, soob)

        return carry

    lax.fori_loop(0, (n + 1) // 2, pair, 0)
    wait_out(ra, sooa)

    @pl.when(n > 1)
    def _():
        wait_out(rb, soob)


def _mesh():
    return plsc.VectorSubcoreMesh(
        core_axis_name="c", subcore_axis_name="s", num_cores=NC, num_subcores=NS
    )


def _tile_scratch():
    return [
        pltpu.VMEM((FEAT, BLK), jnp.float32),
        pltpu.VMEM((BLK // 8, 128), jnp.float32),
        pltpu.VMEM((FEAT, BLK), jnp.float32),
        pltpu.VMEM((BLK // 8, 128), jnp.float32),
        pltpu.SemaphoreType.DMA,
        pltpu.SemaphoreType.DMA,
        pltpu.SemaphoreType.DMA,
        pltpu.SemaphoreType.DMA,
    ]


@jax.jit
def _permute_gather(z, permute):
    zt = z.T

    z_rows128 = pl.kernel(
        _detile_body,
        out_type=jax.ShapeDtypeStruct((Z_DIM // 8, 128), jnp.float32),
        mesh=_mesh(),
        scratch_types=_tile_scratch(),
        compiler_params=pltpu.CompilerParams(use_tc_tiling_on_sc=True, needs_layout_passes=False, disable_bounds_checks=True),
    )(zt)

    out_lin = pl.kernel(
        _gather_body,
        out_type=jax.ShapeDtypeStruct((Z_DIM, FEAT), jnp.float32),
        mesh=_mesh(),
        scratch_types=[
            pltpu.VMEM((CHUNK,), jnp.int32),
            pltpu.VMEM((CHUNK, FEAT), jnp.float32),
            pltpu.VMEM((CHUNK,), jnp.int32),
            pltpu.VMEM((CHUNK, FEAT), jnp.float32),
            pltpu.SemaphoreType.DMA,
            pltpu.SemaphoreType.DMA,
            pltpu.SemaphoreType.DMA,
            pltpu.SemaphoreType.DMA,
            pltpu.SemaphoreType.DMA,
            pltpu.SemaphoreType.DMA,
        ],
        compiler_params=pltpu.CompilerParams(use_tc_tiling_on_sc=False),
    )(z_rows128.reshape(Z_DIM, FEAT), permute)

    out_t = pl.kernel(
        _retile_body,
        out_type=jax.ShapeDtypeStruct((FEAT, Z_DIM), jnp.float32),
        mesh=_mesh(),
        scratch_types=[
            pltpu.VMEM((FEAT, BLK), jnp.float32),
            pltpu.VMEM((BLK // 8, 128), jnp.float32),
            pltpu.VMEM((FEAT, BLK), jnp.float32),
            pltpu.VMEM((BLK // 8, 128), jnp.float32),
            pltpu.SemaphoreType.DMA,
            pltpu.SemaphoreType.DMA,
            pltpu.SemaphoreType.DMA,
            pltpu.SemaphoreType.DMA,
        ],
        compiler_params=pltpu.CompilerParams(use_tc_tiling_on_sc=True, needs_layout_passes=False, disable_bounds_checks=True),
    )(out_lin.reshape(Z_DIM // 8, 128))

    return out_t.T


def kernel(z, permute):
    return _permute_gather(z, permute.astype(jnp.int32))


# speedup vs baseline: 1.1485x; 1.0005x over previous
import jax
import jax.numpy as jnp
from jax import lax
from jax.experimental import pallas as pl
from jax.experimental.pallas import tpu as pltpu
from jax.experimental.pallas import tpu_sc as plsc

Z_DIM = 1_000_000
FEAT = 16
NC = 2
NS = 16
NW = NC * NS

BLK = 1024
NFULL = Z_DIM // BLK
TAIL_A = 512
NB_LO = NFULL // NW
NB_REM = NFULL - NB_LO * NW

CHUNK = 2000
NCHUNK = Z_DIM // CHUNK
BASE_PER_W = NCHUNK // NW
REM = NCHUNK - BASE_PER_W * NW


def _worker_id():
    return lax.axis_index("s") * NC + lax.axis_index("c")


def _shuffle_detile(t01, rv, ncols):
    iota = lax.broadcasted_iota(jnp.int32, (16,), 0)
    mjs = [lax.bitwise_and(iota + j, 15) for j in range(16)]

    def body(g, carry):
        c0 = g * 16
        cvec = c0 + iota
        base16 = lax.shift_left(cvec, 4)
        vals = [plsc.load_gather(t01, [mjs[j], cvec]) for j in range(16)]
        for j in range(16):
            f = base16 + mjs[j]
            plsc.store_scatter(
                rv, [lax.shift_right_logical(f, 7), lax.bitwise_and(f, 127)],
                vals[j])
        return carry

    lax.fori_loop(0, ncols // 16, body, 0, unroll=4)


def _shuffle_retile(u, rv, ncols):
    iota = lax.broadcasted_iota(jnp.int32, (16,), 0)
    mjs = [lax.bitwise_and(iota + j, 15) for j in range(16)]

    def body(g, carry):
        c0 = g * 16
        cvec = c0 + iota
        base16 = lax.shift_left(cvec, 4)
        flats = [base16 + mjs[j] for j in range(16)]
        vals = [
            plsc.load_gather(
                rv, [lax.shift_right_logical(f, 7), lax.bitwise_and(f, 127)])
            for f in flats
        ]
        for j in range(16):
            plsc.store_scatter(u, [mjs[j], cvec], vals[j])
        return carry

    lax.fori_loop(0, ncols // 16, body, 0, unroll=4)


def _detile_body(zt_hbm, out_hbm, t01a, rva, t01b, rvb,
                 sia, sib, soa, sob):
    w = _worker_id()
    nb = jnp.where(w < NB_REM, NB_LO + 1, NB_LO)

    def issue_in(cb, t01, sem):
        col = cb * BLK
        pltpu.async_copy(zt_hbm.at[pl.ds(0, 8), pl.ds(col, BLK)],
                         t01.at[pl.ds(0, 8)], sem)
        pltpu.async_copy(zt_hbm.at[pl.ds(8, 8), pl.ds(col, BLK)],
                         t01.at[pl.ds(8, 8)], sem)

    def wait_in(t01, sem):
        pltpu.make_async_copy(zt_hbm.at[pl.ds(0, 8), pl.ds(0, BLK)],
                              t01.at[pl.ds(0, 8)], sem).wait()
        pltpu.make_async_copy(zt_hbm.at[pl.ds(8, 8), pl.ds(0, BLK)],
                              t01.at[pl.ds(8, 8)], sem).wait()

    def issue_out(cb, rv, sem):
        pltpu.async_copy(rv, out_hbm.at[pl.ds(cb * (BLK // 8), BLK // 8)], sem)

    def wait_out(rv, sem):
        pltpu.make_async_copy(rv, out_hbm.at[pl.ds(0, BLK // 8)], sem).wait()

    @pl.when(nb > 0)
    def _():
        issue_in(w, t01a, sia)

        def pair(k, carry):
            i0 = 2 * k
            cb0 = w + 32 * i0

            @pl.when(i0 + 1 < nb)
            def _():
                issue_in(w + 32 * (i0 + 1), t01b, sib)

            wait_in(t01a, sia)
            _shuffle_detile(t01a, rva, BLK)

            @pl.when(k > 0)
            def _():
                wait_out(rva, soa)

            issue_out(cb0, rva, soa)

            @pl.when(i0 + 1 < nb)
            def _():
                @pl.when(i0 + 2 < nb)
                def _():
                    issue_in(w + 32 * (i0 + 2), t01a, sia)

                wait_in(t01b, sib)
                _shuffle_detile(t01b, rvb, BLK)

                @pl.when(k > 0)
                def _():
                    wait_out(rvb, sob)

                issue_out(w + 32 * (i0 + 1), rvb, sob)

            return carry

        npairs = (nb + 1) // 2
        lax.fori_loop(0, npairs, pair, 0)
        wait_out(rva, soa)

        @pl.when(nb > 1)
        def _():
            wait_out(rvb, sob)

    @pl.when(w == 30)
    def _():
        col = NFULL * BLK
        pltpu.async_copy(zt_hbm.at[pl.ds(0, 8), pl.ds(col, TAIL_A)],
                         t01a.at[pl.ds(0, 8), pl.ds(0, TAIL_A)], sia)
        pltpu.async_copy(zt_hbm.at[pl.ds(8, 8), pl.ds(col, TAIL_A)],
                         t01a.at[pl.ds(8, 8), pl.ds(0, TAIL_A)], sia)
        pltpu.make_async_copy(zt_hbm.at[pl.ds(0, 8), pl.ds(0, TAIL_A)],
                              t01a.at[pl.ds(0, 8), pl.ds(0, TAIL_A)], sia).wait()
        pltpu.make_async_copy(zt_hbm.at[pl.ds(8, 8), pl.ds(0, TAIL_A)],
                              t01a.at[pl.ds(8, 8), pl.ds(0, TAIL_A)], sia).wait()
        _shuffle_detile(t01a, rva, TAIL_A)
        pltpu.async_copy(rva.at[pl.ds(0, TAIL_A // 8)],
                         out_hbm.at[pl.ds(col // 8, TAIL_A // 8)], soa)
        pltpu.make_async_copy(rva.at[pl.ds(0, TAIL_A // 8)],
                              out_hbm.at[pl.ds(0, TAIL_A // 8)], soa).wait()

    @pl.when(w == 31)
    def _():
        col = pl.multiple_of(jnp.int32(Z_DIM - 64), 128)
        colw = Z_DIM - 64
        pltpu.async_copy(zt_hbm.at[pl.ds(0, 8), pl.ds(col, 128)],
                         t01a.at[pl.ds(0, 8), pl.ds(0, 128)], sia)
        pltpu.async_copy(zt_hbm.at[pl.ds(8, 8), pl.ds(col, 128)],
                         t01a.at[pl.ds(8, 8), pl.ds(0, 128)], sia)
        pltpu.make_async_copy(zt_hbm.at[pl.ds(0, 8), pl.ds(0, 128)],
                              t01a.at[pl.ds(0, 8), pl.ds(0, 128)], sia).wait()
        pltpu.make_async_copy(zt_hbm.at[pl.ds(8, 8), pl.ds(0, 128)],
                              t01a.at[pl.ds(8, 8), pl.ds(0, 128)], sia).wait()
        _shuffle_detile(t01a, rva, 128)
        pltpu.async_copy(rva.at[pl.ds(0, 8)],
                         out_hbm.at[pl.ds(colw // 8, 8)], soa)
        pltpu.make_async_copy(rva.at[pl.ds(0, 8)],
                              out_hbm.at[pl.ds(0, 8)], soa).wait()


def _retile_body(rows_hbm, out_hbm, ua, rva, ub, rvb,
                 sia, sib, soa, sob):
    w = _worker_id()
    nb = jnp.where(w < NB_REM, NB_LO + 1, NB_LO)

    def issue_in(cb, rv, sem):
        pltpu.async_copy(rows_hbm.at[pl.ds(cb * (BLK // 8), BLK // 8)], rv, sem)

    def wait_in(rv, sem):
        pltpu.make_async_copy(rows_hbm.at[pl.ds(0, BLK // 8)], rv, sem).wait()

    def issue_out(cb, u, sem):
        col = cb * BLK
        pltpu.async_copy(u.at[pl.ds(0, 8)], out_hbm.at[pl.ds(0, 8), pl.ds(col, BLK)], sem)
        pltpu.async_copy(u.at[pl.ds(8, 8)], out_hbm.at[pl.ds(8, 8), pl.ds(col, BLK)], sem)

    def wait_out(u, sem):
        pltpu.make_async_copy(u.at[pl.ds(0, 8)], out_hbm.at[pl.ds(0, 8), pl.ds(0, BLK)], sem).wait()
        pltpu.make_async_copy(u.at[pl.ds(8, 8)], out_hbm.at[pl.ds(8, 8), pl.ds(0, BLK)], sem).wait()

    @pl.when(nb > 0)
    def _():
        issue_in(w, rva, sia)

        def pair(k, carry):
            i0 = 2 * k
            cb0 = w + 32 * i0

            @pl.when(i0 + 1 < nb)
            def _():
                issue_in(w + 32 * (i0 + 1), rvb, sib)

            wait_in(rva, sia)

            @pl.when(k > 0)
            def _():
                wait_out(ua, soa)

            _shuffle_retile(ua, rva, BLK)
            issue_out(cb0, ua, soa)

            @pl.when(i0 + 1 < nb)
            def _():
                @pl.when(i0 + 2 < nb)
                def _():
                    issue_in(w + 32 * (i0 + 2), rva, sia)

                wait_in(rvb, sib)

                @pl.when(k > 0)
                def _():
                    wait_out(ub, sob)

                _shuffle_retile(ub, rvb, BLK)
                issue_out(w + 32 * (i0 + 1), ub, sob)

            return carry

        npairs = (nb + 1) // 2
        lax.fori_loop(0, npairs, pair, 0)
        wait_out(ua, soa)

        @pl.when(nb > 1)
        def _():
            wait_out(ub, sob)

    @pl.when(w == 30)
    def _():
        col = NFULL * BLK
        pltpu.async_copy(rows_hbm.at[pl.ds(col // 8, TAIL_A // 8)],
                         rva.at[pl.ds(0, TAIL_A // 8)], sia)
        pltpu.make_async_copy(rows_hbm.at[pl.ds(0, TAIL_A // 8)],
                              rva.at[pl.ds(0, TAIL_A // 8)], sia).wait()
        _shuffle_retile(ua, rva, TAIL_A)
        pltpu.async_copy(ua.at[pl.ds(0, 8), pl.ds(0, TAIL_A)],
                         out_hbm.at[pl.ds(0, 8), pl.ds(col, TAIL_A)], soa)
        pltpu.async_copy(ua.at[pl.ds(8, 8), pl.ds(0, TAIL_A)],
                         out_hbm.at[pl.ds(8, 8), pl.ds(col, TAIL_A)], soa)
        pltpu.make_async_copy(ua.at[pl.ds(0, 8), pl.ds(0, TAIL_A)],
                              out_hbm.at[pl.ds(0, 8), pl.ds(0, TAIL_A)], soa).wait()
        pltpu.make_async_copy(ua.at[pl.ds(8, 8), pl.ds(0, TAIL_A)],
                              out_hbm.at[pl.ds(8, 8), pl.ds(0, TAIL_A)], soa).wait()

    @pl.when(w == 31)
    def _():
        colw = Z_DIM - 64
        col = pl.multiple_of(jnp.int32(Z_DIM - 64), 128)
        pltpu.async_copy(rows_hbm.at[pl.ds(colw // 8, 8)],
                         rva.at[pl.ds(0, 8)], sia)
        pltpu.make_async_copy(rows_hbm.at[pl.ds(0, 8)],
                              rva.at[pl.ds(0, 8)], sia).wait()
        _shuffle_retile(ua, rva, 128)
        pltpu.async_copy(ua.at[pl.ds(0, 8), pl.ds(0, 128)],
                         out_hbm.at[pl.ds(0, 8), pl.ds(col, 128)], soa)
        pltpu.async_copy(ua.at[pl.ds(8, 8), pl.ds(0, 128)],
                         out_hbm.at[pl.ds(8, 8), pl.ds(col, 128)], soa)
        pltpu.make_async_copy(ua.at[pl.ds(0, 8), pl.ds(0, 128)],
                              out_hbm.at[pl.ds(0, 8), pl.ds(0, 128)], soa).wait()
        pltpu.make_async_copy(ua.at[pl.ds(8, 8), pl.ds(0, 128)],
                              out_hbm.at[pl.ds(8, 8), pl.ds(0, 128)], soa).wait()


def _gather_body(z_hbm, p_hbm, out_hbm, ia, ra, ib, rb,
                 siia, sga, sooa, siib, sgb, soob):
    w = _worker_id()
    start = BASE_PER_W * w + jnp.minimum(w, REM)
    n = jnp.where(w < REM, BASE_PER_W + 1, BASE_PER_W)

    def issue_idx(c, iv, sem):
        pltpu.async_copy(p_hbm.at[pl.ds(c * CHUNK, CHUNK)], iv, sem)

    def wait_idx(iv, sem):
        pltpu.make_async_copy(p_hbm.at[pl.ds(0, CHUNK)], iv, sem).wait()

    def issue_g(iv, rv, sem):
        pltpu.async_copy(z_hbm.at[iv], rv, sem)

    def wait_g(iv, rv, sem):
        pltpu.make_async_copy(z_hbm.at[iv], rv, sem).wait()

    def issue_out(c, rv, sem):
        pltpu.async_copy(rv, out_hbm.at[pl.ds(c * CHUNK, CHUNK)], sem)

    def wait_out(rv, sem):
        pltpu.make_async_copy(rv, out_hbm.at[pl.ds(0, CHUNK)], sem).wait()

    issue_idx(start, ia, siia)

    def pair(k, carry):
        i0 = 2 * k
        c0 = start + i0

        @pl.when(i0 + 1 < n)
        def _():
            issue_idx(c0 + 1, ib, siib)

        wait_idx(ia, siia)

        @pl.when(k > 0)
        def _():
            wait_out(ra, sooa)

        issue_g(ia, ra, sga)

        @pl.when(i0 + 1 < n)
        def _():
            wait_idx(ib, siib)

            @pl.when(k > 0)
            def _():
                wait_out(rb, soob)

            issue_g(ib, rb, sgb)

        wait_g(ia, ra, sga)
        issue_out(c0, ra, sooa)

        @pl.when(i0 + 2 < n)
        def _():
            issue_idx(c0 + 2, ia, siia)

        @pl.when(i0 + 1 < n)
        def _():
            wait_g(ib, rb, sgb)
            issue_out(c0 + 1, rb, soob)

        return carry

    lax.fori_loop(0, (n + 1) // 2, pair, 0)
    wait_out(ra, sooa)

    @pl.when(n > 1)
    def _():
        wait_out(rb, soob)


def _mesh():
    return plsc.VectorSubcoreMesh(
        core_axis_name="c", subcore_axis_name="s", num_cores=NC, num_subcores=NS
    )


def _tile_scratch():
    return [
        pltpu.VMEM((FEAT, BLK), jnp.float32),
        pltpu.VMEM((BLK // 8, 128), jnp.float32),
        pltpu.VMEM((FEAT, BLK), jnp.float32),
        pltpu.VMEM((BLK // 8, 128), jnp.float32),
        pltpu.SemaphoreType.DMA,
        pltpu.SemaphoreType.DMA,
        pltpu.SemaphoreType.DMA,
        pltpu.SemaphoreType.DMA,
    ]


@jax.jit
def _permute_gather(z, permute):
    zt = z.T

    z_rows128 = pl.kernel(
        _detile_body,
        out_type=jax.ShapeDtypeStruct((Z_DIM // 8, 128), jnp.float32),
        mesh=_mesh(),
        scratch_types=_tile_scratch(),
        compiler_params=pltpu.CompilerParams(use_tc_tiling_on_sc=True, needs_layout_passes=False, disable_bounds_checks=True),
    )(zt)

    out_lin = pl.kernel(
        _gather_body,
        out_type=jax.ShapeDtypeStruct((Z_DIM, FEAT), jnp.float32),
        mesh=_mesh(),
        scratch_types=[
            pltpu.VMEM((CHUNK,), jnp.int32),
            pltpu.VMEM((CHUNK, FEAT), jnp.float32),
            pltpu.VMEM((CHUNK,), jnp.int32),
            pltpu.VMEM((CHUNK, FEAT), jnp.float32),
            pltpu.SemaphoreType.DMA,
            pltpu.SemaphoreType.DMA,
            pltpu.SemaphoreType.DMA,
            pltpu.SemaphoreType.DMA,
            pltpu.SemaphoreType.DMA,
            pltpu.SemaphoreType.DMA,
        ],
        compiler_params=pltpu.CompilerParams(use_tc_tiling_on_sc=False),
    )(z_rows128.reshape(Z_DIM, FEAT), permute)

    out_t = pl.kernel(
        _retile_body,
        out_type=jax.ShapeDtypeStruct((FEAT, Z_DIM), jnp.float32),
        mesh=_mesh(),
        scratch_types=[
            pltpu.VMEM((FEAT, BLK), jnp.float32),
            pltpu.VMEM((BLK // 8, 128), jnp.float32),
            pltpu.VMEM((FEAT, BLK), jnp.float32),
            pltpu.VMEM((BLK // 8, 128), jnp.float32),
            pltpu.SemaphoreType.DMA,
            pltpu.SemaphoreType.DMA,
            pltpu.SemaphoreType.DMA,
            pltpu.SemaphoreType.DMA,
        ],
        compiler_params=pltpu.CompilerParams(use_tc_tiling_on_sc=True, needs_layout_passes=False, disable_bounds_checks=True),
    )(out_lin.reshape(Z_DIM // 8, 128))

    return out_t.T


def kernel(z, permute):
    return _permute_gather(z, permute.astype(jnp.int32))
